# Initial kernel scaffold; baseline (speedup 1.0000x reference)
import jax
import jax.numpy as jnp
from jax.experimental import pallas as pl


def kernel(rois, scores, max_output_size):
    raise NotImplementedError("write your pallas kernel here")


# speedup vs baseline: 34.1231x; 34.1231x over previous
import jax
import jax.numpy as jnp
from jax.experimental import pallas as pl
from jax.experimental.pallas import tpu as pltpu

_N = 20000
_R, _C = 8, 2560
_NP = _R * _C
_MAXO = 1000
_NEG = float(jnp.float32(-1e30))
_THR = float(jnp.float32(0.7))


def _nms_body(mos_ref, s_ref, y1_ref, x1_ref, y2_ref, x2_ref, ar_ref,
              out_ref, sw_ref):
    sw_ref[...] = s_ref[...]
    mos = mos_ref[0]
    rows = jax.lax.broadcasted_iota(jnp.int32, (_R, _C), 0)
    cols = jax.lax.broadcasted_iota(jnp.int32, (_R, _C), 1)
    idxg = (rows * _C + cols).astype(jnp.float32)

    b0y1 = y1_ref[0, 0]
    b0x1 = x1_ref[0, 0]
    b0y2 = y2_ref[0, 0]
    b0x2 = x2_ref[0, 0]

    def body(i, _):
        s = sw_ref[...]
        m = jnp.max(s)
        idx = jnp.min(jnp.where(s == m, idxg, jnp.float32(_NP)))
        valid = (m > _NEG / 2.0) & (i < mos)
        sel = idxg == idx
        by1 = jnp.max(jnp.where(sel, y1_ref[...], _NEG))
        bx1 = jnp.max(jnp.where(sel, x1_ref[...], _NEG))
        by2 = jnp.max(jnp.where(sel, y2_ref[...], _NEG))
        bx2 = jnp.max(jnp.where(sel, x2_ref[...], _NEG))
        barea = jnp.max(jnp.where(sel, ar_ref[...], _NEG))

        yy1 = jnp.maximum(by1, y1_ref[...])
        xx1 = jnp.maximum(bx1, x1_ref[...])
        yy2 = jnp.minimum(by2, y2_ref[...])
        xx2 = jnp.minimum(bx2, x2_ref[...])
        inter = jnp.maximum(yy2 - yy1, 0.0) * jnp.maximum(xx2 - xx1, 0.0)
        union = barea + ar_ref[...] - inter
        iou = jnp.where(union > 0.0, inter / union, 0.0)
        supp = (iou >= _THR) | (idxg == idx)
        sw_ref[...] = jnp.where(supp, _NEG, s)

        oy1 = jnp.where(valid, by1, b0y1)
        ox1 = jnp.where(valid, bx1, b0x1)
        oy2 = jnp.where(valid, by2, b0y2)
        ox2 = jnp.where(valid, bx2, b0x2)
        out_ref[pl.ds(i, 1), pl.ds(0, 1)] = jnp.full((1, 1), oy1)
        out_ref[pl.ds(i, 1), pl.ds(1, 1)] = jnp.full((1, 1), ox1)
        out_ref[pl.ds(i, 1), pl.ds(2, 1)] = jnp.full((1, 1), oy2)
        out_ref[pl.ds(i, 1), pl.ds(3, 1)] = jnp.full((1, 1), ox2)
        return 0

    jax.lax.fori_loop(0, _MAXO, body, 0)


def kernel(rois, scores, max_output_size):
    s = jnp.squeeze(scores, axis=-1)
    s = jnp.concatenate([s, jnp.full((_NP - _N,), _NEG, jnp.float32)])
    planes = []
    for k in range(4):
        p = jnp.concatenate([rois[:, k], jnp.zeros((_NP - _N,), jnp.float32)])
        planes.append(p.reshape(_R, _C))
    areas = (jnp.maximum(planes[2] - planes[0], 0.0)
             * jnp.maximum(planes[3] - planes[1], 0.0))
    mos = jnp.asarray(max_output_size, jnp.int32).reshape(1)

    out = pl.pallas_call(
        _nms_body,
        out_shape=jax.ShapeDtypeStruct((_MAXO, 4), jnp.float32),
        in_specs=[
            pl.BlockSpec(memory_space=pltpu.SMEM),
            pl.BlockSpec(memory_space=pltpu.VMEM),
            pl.BlockSpec(memory_space=pltpu.VMEM),
            pl.BlockSpec(memory_space=pltpu.VMEM),
            pl.BlockSpec(memory_space=pltpu.VMEM),
            pl.BlockSpec(memory_space=pltpu.VMEM),
            pl.BlockSpec(memory_space=pltpu.VMEM),
        ],
        scratch_shapes=[pltpu.VMEM((_R, _C), jnp.float32)],
    )(mos, s.reshape(_R, _C), planes[0], planes[1], planes[2], planes[3],
      areas)
    return out


# speedup vs baseline: 37.6800x; 1.1042x over previous
import functools

import numpy as np

import jax
import jax.numpy as jnp
from jax import lax
from jax.experimental import pallas as pl
from jax.experimental.pallas import tpu as pltpu
from jax.experimental.pallas import tpu_sc as plsc

_N = 20000
_NP = 20480
_NW = 32
_PT = _NP // _NW
_NG = _PT // 64
_M = _NG * 16
_NC = _NW * _M
_CR, _CC = 8, _NC // 8
_R, _C = 8, 2560
_MAXO = 1000
_NEG = float(np.float32(-1e30))


def _lexgt(s, i, sp, ip):
    return (s > sp) | ((s == sp) & (i < ip))


def _ce_stage(lane, s, idx, j, want_max):
    perm = lane ^ j
    sp = s[perm]
    ip = idx[perm]
    gt = _lexgt(s, idx, sp, ip)
    gi = jnp.where(gt, 1, 0)
    wi = jnp.where(want_max, 1, 0)
    take_self = (gi ^ wi) == 0
    return jnp.where(take_self, s, sp), jnp.where(take_self, idx, ip)


def _sort16_desc(lane, s, idx):
    for k in (2, 4, 8, 16):
        j = k // 2
        while j >= 1:
            jb = (lane >> j.bit_length() - 1) & 1
            kb = (lane >> k.bit_length() - 1) & 1
            want_max = jb == kb
            s, idx = _ce_stage(lane, s, idx, j, want_max)
            j //= 2
    return s, idx


def _merge_top16(lane, sa, ia, sb, ib):
    rperm = lane ^ 15
    rb = sb[rperm]
    rib = ib[rperm]
    gt = _lexgt(sa, ia, rb, rib)
    ts = jnp.where(gt, sa, rb)
    ti = jnp.where(gt, ia, rib)
    bs = jnp.where(gt, rb, sa)
    j = 8
    while j >= 1:
        want_max = (lane & j) == 0
        ts, ti = _ce_stage(lane, ts, ti, j, want_max)
        j //= 2
    bmax = bs
    for d in (8, 4, 2, 1):
        bmax = jnp.maximum(bmax, bmax[lane ^ d])
    return ts, ti, bmax


def _build_sc_select():
  mesh = plsc.VectorSubcoreMesh(core_axis_name="c", subcore_axis_name="s")

  @functools.partial(
    pl.kernel,
    mesh=mesh,
    out_type=[
        jax.ShapeDtypeStruct((_NC,), jnp.float32),
        jax.ShapeDtypeStruct((_NC,), jnp.int32),
        jax.ShapeDtypeStruct((_NC,), jnp.float32),
        jax.ShapeDtypeStruct((_NC,), jnp.float32),
        jax.ShapeDtypeStruct((_NC,), jnp.float32),
        jax.ShapeDtypeStruct((_NC,), jnp.float32),
        jax.ShapeDtypeStruct((_NW * 16,), jnp.float32),
    ],
    scratch_types=[
        pltpu.VMEM((_PT,), jnp.float32),
        pltpu.VMEM((_PT,), jnp.float32),
        pltpu.VMEM((_PT,), jnp.float32),
        pltpu.VMEM((_PT,), jnp.float32),
        pltpu.VMEM((_PT,), jnp.float32),
        pltpu.VMEM((_M,), jnp.float32),
        pltpu.VMEM((_M,), jnp.int32),
        pltpu.VMEM((_M,), jnp.float32),
        pltpu.VMEM((_M,), jnp.float32),
        pltpu.VMEM((_M,), jnp.float32),
        pltpu.VMEM((_M,), jnp.float32),
        pltpu.VMEM((16,), jnp.float32),
    ],
  )
  def _sc_sel(s_hbm, y1_hbm, x1_hbm, y2_hbm, x2_hbm,
              cs_out, ci_out, cy1_out, cx1_out, cy2_out, cx2_out, thr_out,
              s_v, y1_v, x1_v, y2_v, x2_v,
              cs_v, ci_v, cy1_v, cx1_v, cy2_v, cx2_v, thr_v):
    wid = lax.axis_index("s") * 2 + lax.axis_index("c")
    base = wid * _PT
    pltpu.sync_copy(s_hbm.at[pl.ds(base, _PT)], s_v)
    pltpu.sync_copy(y1_hbm.at[pl.ds(base, _PT)], y1_v)
    pltpu.sync_copy(x1_hbm.at[pl.ds(base, _PT)], x1_v)
    pltpu.sync_copy(y2_hbm.at[pl.ds(base, _PT)], y2_v)
    pltpu.sync_copy(x2_hbm.at[pl.ds(base, _PT)], x2_v)

    lane = lax.iota(jnp.int32, 16)
    neg = jnp.float32(_NEG)

    def _group(g, thrmax):
        goff = g * 64
        vs, vi = [], []
        for k in range(4):
            sv = s_v[pl.ds(goff + k * 16, 16)]
            gi = base + goff + k * 16 + lane
            ss, si = _sort16_desc(lane, sv, gi)
            vs.append(ss)
            vi.append(si)
        t01s, t01i, b01 = _merge_top16(lane, vs[0], vi[0], vs[1], vi[1])
        t23s, t23i, b23 = _merge_top16(lane, vs[2], vi[2], vs[3], vi[3])
        ts, ti, bt = _merge_top16(lane, t01s, t01i, t23s, t23i)
        thr_g = jnp.maximum(jnp.maximum(b01, b23), bt)
        thrmax = jnp.maximum(thrmax, thr_g)

        li = ti - (base + goff)
        vno = li >> 4
        lno = li - (vno << 4)
        outs = []
        for c_v in (y1_v, x1_v, y2_v, x2_v):
            r = jnp.zeros((16,), jnp.float32)
            for k in range(4):
                ck = c_v[pl.ds(goff + k * 16, 16)]
                r = jnp.where(vno == k, ck[lno], r)
            outs.append(r)

        cs_v[pl.ds(g * 16, 16)] = ts
        ci_v[pl.ds(g * 16, 16)] = ti
        cy1_v[pl.ds(g * 16, 16)] = outs[0]
        cx1_v[pl.ds(g * 16, 16)] = outs[1]
        cy2_v[pl.ds(g * 16, 16)] = outs[2]
        cx2_v[pl.ds(g * 16, 16)] = outs[3]
        return thrmax

    thrmax = lax.fori_loop(0, _NG, _group, jnp.full((16,), neg))
    thr_v[...] = thrmax

    pltpu.sync_copy(cs_v, cs_out.at[pl.ds(wid * _M, _M)])
    pltpu.sync_copy(ci_v, ci_out.at[pl.ds(wid * _M, _M)])
    pltpu.sync_copy(cy1_v, cy1_out.at[pl.ds(wid * _M, _M)])
    pltpu.sync_copy(cx1_v, cx1_out.at[pl.ds(wid * _M, _M)])
    pltpu.sync_copy(cy2_v, cy2_out.at[pl.ds(wid * _M, _M)])
    pltpu.sync_copy(cx2_v, cx2_out.at[pl.ds(wid * _M, _M)])
    pltpu.sync_copy(thr_v, thr_out.at[pl.ds(wid * 16, 16)])

  return _sc_sel


_sc_select_cached = None


def _sc_select(*args):
    global _sc_select_cached
    if _sc_select_cached is None:
        _sc_select_cached = _build_sc_select()
    return _sc_select_cached(*args)


def _nms_cand_body(mos_ref, b0_ref, thr_ref, s_ref, idxf_ref,
                   y1_ref, x1_ref, y2_ref, x2_ref,
                   out_ref, flag_ref, sw_ref, ar_ref):
    sw_ref[...] = s_ref[...]
    ar_ref[...] = (jnp.maximum(y2_ref[...] - y1_ref[...], 0.0)
                   * jnp.maximum(x2_ref[...] - x1_ref[...], 0.0))
    flag_ref[0] = 0
    mos = mos_ref[0]
    mthr = jnp.max(thr_ref[...])
    b0y1 = b0_ref[0]
    b0x1 = b0_ref[1]
    b0y2 = b0_ref[2]
    b0x2 = b0_ref[3]

    def body(i, _):
        s = sw_ref[...]
        idxp = idxf_ref[...]
        m = jnp.max(s)
        flag_ref[0] = flag_ref[0] | (m <= mthr).astype(jnp.int32)
        idx = jnp.min(jnp.where(s == m, idxp, jnp.float32(_NP)))
        valid = (m > _NEG / 2.0) & (i < mos)
        sel = idxp == idx
        by1 = jnp.max(jnp.where(sel, y1_ref[...], _NEG))
        bx1 = jnp.max(jnp.where(sel, x1_ref[...], _NEG))
        by2 = jnp.max(jnp.where(sel, y2_ref[...], _NEG))
        bx2 = jnp.max(jnp.where(sel, x2_ref[...], _NEG))
        barea = jnp.max(jnp.where(sel, ar_ref[...], _NEG))

        yy1 = jnp.maximum(by1, y1_ref[...])
        xx1 = jnp.maximum(bx1, x1_ref[...])
        yy2 = jnp.minimum(by2, y2_ref[...])
        xx2 = jnp.minimum(bx2, x2_ref[...])
        inter = jnp.maximum(yy2 - yy1, 0.0) * jnp.maximum(xx2 - xx1, 0.0)
        union = barea + ar_ref[...] - inter
        iou = jnp.where(union > 0.0, inter / union, 0.0)
        supp = (iou >= 0.7) | sel
        sw_ref[...] = jnp.where(supp, _NEG, s)

        oy1 = jnp.where(valid, by1, b0y1)
        ox1 = jnp.where(valid, bx1, b0x1)
        oy2 = jnp.where(valid, by2, b0y2)
        ox2 = jnp.where(valid, bx2, b0x2)
        out_ref[pl.ds(i, 1), pl.ds(0, 1)] = jnp.full((1, 1), oy1)
        out_ref[pl.ds(i, 1), pl.ds(1, 1)] = jnp.full((1, 1), ox1)
        out_ref[pl.ds(i, 1), pl.ds(2, 1)] = jnp.full((1, 1), oy2)
        out_ref[pl.ds(i, 1), pl.ds(3, 1)] = jnp.full((1, 1), ox2)
        return 0

    lax.fori_loop(0, _MAXO, body, 0)


def _nms_full_body(mos_ref, s_ref, y1_ref, x1_ref, y2_ref, x2_ref,
                   out_ref, sw_ref, ar_ref):
    sw_ref[...] = s_ref[...]
    ar_ref[...] = (jnp.maximum(y2_ref[...] - y1_ref[...], 0.0)
                   * jnp.maximum(x2_ref[...] - x1_ref[...], 0.0))
    mos = mos_ref[0]
    rows = lax.broadcasted_iota(jnp.int32, (_R, _C), 0)
    cols = lax.broadcasted_iota(jnp.int32, (_R, _C), 1)
    idxg = (rows * _C + cols).astype(jnp.float32)

    b0y1 = y1_ref[0, 0]
    b0x1 = x1_ref[0, 0]
    b0y2 = y2_ref[0, 0]
    b0x2 = x2_ref[0, 0]

    def body(i, _):
        s = sw_ref[...]
        m = jnp.max(s)
        idx = jnp.min(jnp.where(s == m, idxg, jnp.float32(_NP)))
        valid = (m > _NEG / 2.0) & (i < mos)
        sel = idxg == idx
        by1 = jnp.max(jnp.where(sel, y1_ref[...], _NEG))
        bx1 = jnp.max(jnp.where(sel, x1_ref[...], _NEG))
        by2 = jnp.max(jnp.where(sel, y2_ref[...], _NEG))
        bx2 = jnp.max(jnp.where(sel, x2_ref[...], _NEG))
        barea = jnp.max(jnp.where(sel, ar_ref[...], _NEG))

        yy1 = jnp.maximum(by1, y1_ref[...])
        xx1 = jnp.maximum(bx1, x1_ref[...])
        yy2 = jnp.minimum(by2, y2_ref[...])
        xx2 = jnp.minimum(bx2, x2_ref[...])
        inter = jnp.maximum(yy2 - yy1, 0.0) * jnp.maximum(xx2 - xx1, 0.0)
        union = barea + ar_ref[...] - inter
        iou = jnp.where(union > 0.0, inter / union, 0.0)
        supp = (iou >= 0.7) | sel
        sw_ref[...] = jnp.where(supp, _NEG, s)

        oy1 = jnp.where(valid, by1, b0y1)
        ox1 = jnp.where(valid, bx1, b0x1)
        oy2 = jnp.where(valid, by2, b0y2)
        ox2 = jnp.where(valid, bx2, b0x2)
        out_ref[pl.ds(i, 1), pl.ds(0, 1)] = jnp.full((1, 1), oy1)
        out_ref[pl.ds(i, 1), pl.ds(1, 1)] = jnp.full((1, 1), ox1)
        out_ref[pl.ds(i, 1), pl.ds(2, 1)] = jnp.full((1, 1), oy2)
        out_ref[pl.ds(i, 1), pl.ds(3, 1)] = jnp.full((1, 1), ox2)
        return 0

    lax.fori_loop(0, _MAXO, body, 0)


def kernel(rois, scores, max_output_size):
    s = jnp.squeeze(scores, axis=-1)
    s_p = jnp.concatenate([s, jnp.full((_NP - _N,), _NEG, jnp.float32)])
    zpad = jnp.zeros((_NP - _N,), jnp.float32)
    y1 = jnp.concatenate([rois[:, 0], zpad])
    x1 = jnp.concatenate([rois[:, 1], zpad])
    y2 = jnp.concatenate([rois[:, 2], zpad])
    x2 = jnp.concatenate([rois[:, 3], zpad])
    mos = jnp.asarray(max_output_size, jnp.int32).reshape(1)
    b0 = rois[0]

    cs, ci, cy1, cx1, cy2, cx2, thr = _sc_select(s_p, y1, x1, y2, x2)

    vspec = pl.BlockSpec(memory_space=pltpu.VMEM)
    sspec = pl.BlockSpec(memory_space=pltpu.SMEM)
    fast_out, flag = pl.pallas_call(
        _nms_cand_body,
        out_shape=[jax.ShapeDtypeStruct((_MAXO, 4), jnp.float32),
                   jax.ShapeDtypeStruct((1,), jnp.int32)],
        in_specs=[sspec, sspec] + [vspec] * 7,
        out_specs=[vspec, sspec],
        scratch_shapes=[pltpu.VMEM((_CR, _CC), jnp.float32),
                        pltpu.VMEM((_CR, _CC), jnp.float32)],
    )(mos, b0, thr.reshape(8, -1), cs.reshape(_CR, _CC),
      ci.astype(jnp.float32).reshape(_CR, _CC), cy1.reshape(_CR, _CC),
      cx1.reshape(_CR, _CC), cy2.reshape(_CR, _CC), cx2.reshape(_CR, _CC))

    def _full(_):
        return pl.pallas_call(
            _nms_full_body,
            out_shape=jax.ShapeDtypeStruct((_MAXO, 4), jnp.float32),
            in_specs=[sspec] + [vspec] * 5,
            scratch_shapes=[pltpu.VMEM((_R, _C), jnp.float32),
                            pltpu.VMEM((_R, _C), jnp.float32)],
        )(mos, s_p.reshape(_R, _C), y1.reshape(_R, _C), x1.reshape(_R, _C),
          y2.reshape(_R, _C), x2.reshape(_R, _C))

    def _fast(_):
        return fast_out

    return lax.cond(flag[0] > 0, _full, _fast, None)


# speedup vs baseline: 64.0689x; 1.7003x over previous
import functools

import numpy as np

import jax
import jax.numpy as jnp
from jax import lax
from jax.experimental import pallas as pl
from jax.experimental.pallas import tpu as pltpu
from jax.experimental.pallas import tpu_sc as plsc

_N = 20000
_NP = 20480
_NW = 32
_PT = _NP // _NW
_NG = _PT // 64
_M = _NG * 16
_NC = _NW * _M
_CR, _CC = 8, _NC // 8
_R, _C = 8, 2560
_MAXO = 1000
_NEG = float(np.float32(-1e30))


def _lexgt(s, i, sp, ip):
    return (s > sp) | ((s == sp) & (i < ip))


def _ce_stage(lane, s, idx, j, want_max):
    perm = lane ^ j
    sp = s[perm]
    ip = idx[perm]
    gt = _lexgt(s, idx, sp, ip)
    gi = jnp.where(gt, 1, 0)
    wi = jnp.where(want_max, 1, 0)
    take_self = (gi ^ wi) == 0
    return jnp.where(take_self, s, sp), jnp.where(take_self, idx, ip)


def _sort16_desc(lane, s, idx):
    for k in (2, 4, 8, 16):
        j = k // 2
        while j >= 1:
            jb = (lane >> j.bit_length() - 1) & 1
            kb = (lane >> k.bit_length() - 1) & 1
            want_max = jb == kb
            s, idx = _ce_stage(lane, s, idx, j, want_max)
            j //= 2
    return s, idx


def _merge_top16(lane, sa, ia, sb, ib):
    rperm = lane ^ 15
    rb = sb[rperm]
    rib = ib[rperm]
    gt = _lexgt(sa, ia, rb, rib)
    ts = jnp.where(gt, sa, rb)
    ti = jnp.where(gt, ia, rib)
    bs = jnp.where(gt, rb, sa)
    j = 8
    while j >= 1:
        want_max = (lane & j) == 0
        ts, ti = _ce_stage(lane, ts, ti, j, want_max)
        j //= 2
    bmax = bs
    for d in (8, 4, 2, 1):
        bmax = jnp.maximum(bmax, bmax[lane ^ d])
    return ts, ti, bmax


def _build_sc_select():
  mesh = plsc.VectorSubcoreMesh(core_axis_name="c", subcore_axis_name="s")

  @functools.partial(
    pl.kernel,
    mesh=mesh,
    out_type=[
        jax.ShapeDtypeStruct((_NC,), jnp.float32),
        jax.ShapeDtypeStruct((_NC,), jnp.int32),
        jax.ShapeDtypeStruct((_NC,), jnp.float32),
        jax.ShapeDtypeStruct((_NC,), jnp.float32),
        jax.ShapeDtypeStruct((_NC,), jnp.float32),
        jax.ShapeDtypeStruct((_NC,), jnp.float32),
        jax.ShapeDtypeStruct((_NW * 16,), jnp.float32),
    ],
    scratch_types=[
        pltpu.VMEM((_PT,), jnp.float32),
        pltpu.VMEM((_PT,), jnp.float32),
        pltpu.VMEM((_PT,), jnp.float32),
        pltpu.VMEM((_PT,), jnp.float32),
        pltpu.VMEM((_PT,), jnp.float32),
        pltpu.VMEM((_M,), jnp.float32),
        pltpu.VMEM((_M,), jnp.int32),
        pltpu.VMEM((_M,), jnp.float32),
        pltpu.VMEM((_M,), jnp.float32),
        pltpu.VMEM((_M,), jnp.float32),
        pltpu.VMEM((_M,), jnp.float32),
        pltpu.VMEM((16,), jnp.float32),
    ],
  )
  def _sc_sel(s_hbm, y1_hbm, x1_hbm, y2_hbm, x2_hbm,
              cs_out, ci_out, cy1_out, cx1_out, cy2_out, cx2_out, thr_out,
              s_v, y1_v, x1_v, y2_v, x2_v,
              cs_v, ci_v, cy1_v, cx1_v, cy2_v, cx2_v, thr_v):
    wid = lax.axis_index("s") * 2 + lax.axis_index("c")
    base = wid * _PT
    pltpu.sync_copy(s_hbm.at[pl.ds(base, _PT)], s_v)
    pltpu.sync_copy(y1_hbm.at[pl.ds(base, _PT)], y1_v)
    pltpu.sync_copy(x1_hbm.at[pl.ds(base, _PT)], x1_v)
    pltpu.sync_copy(y2_hbm.at[pl.ds(base, _PT)], y2_v)
    pltpu.sync_copy(x2_hbm.at[pl.ds(base, _PT)], x2_v)

    lane = lax.iota(jnp.int32, 16)
    neg = jnp.float32(_NEG)

    def _group(g, thrmax):
        goff = g * 64
        vs, vi = [], []
        for k in range(4):
            sv = s_v[pl.ds(goff + k * 16, 16)]
            gi = base + goff + k * 16 + lane
            ss, si = _sort16_desc(lane, sv, gi)
            vs.append(ss)
            vi.append(si)
        t01s, t01i, b01 = _merge_top16(lane, vs[0], vi[0], vs[1], vi[1])
        t23s, t23i, b23 = _merge_top16(lane, vs[2], vi[2], vs[3], vi[3])
        ts, ti, bt = _merge_top16(lane, t01s, t01i, t23s, t23i)
        thr_g = jnp.maximum(jnp.maximum(b01, b23), bt)
        thrmax = jnp.maximum(thrmax, thr_g)

        li = ti - (base + goff)
        vno = li >> 4
        lno = li - (vno << 4)
        outs = []
        for c_v in (y1_v, x1_v, y2_v, x2_v):
            r = jnp.zeros((16,), jnp.float32)
            for k in range(4):
                ck = c_v[pl.ds(goff + k * 16, 16)]
                r = jnp.where(vno == k, ck[lno], r)
            outs.append(r)

        cs_v[pl.ds(g * 16, 16)] = ts
        ci_v[pl.ds(g * 16, 16)] = ti
        cy1_v[pl.ds(g * 16, 16)] = outs[0]
        cx1_v[pl.ds(g * 16, 16)] = outs[1]
        cy2_v[pl.ds(g * 16, 16)] = outs[2]
        cx2_v[pl.ds(g * 16, 16)] = outs[3]
        return thrmax

    thrmax = lax.fori_loop(0, _NG, _group, jnp.full((16,), neg))
    thr_v[...] = thrmax

    pltpu.sync_copy(cs_v, cs_out.at[pl.ds(wid * _M, _M)])
    pltpu.sync_copy(ci_v, ci_out.at[pl.ds(wid * _M, _M)])
    pltpu.sync_copy(cy1_v, cy1_out.at[pl.ds(wid * _M, _M)])
    pltpu.sync_copy(cx1_v, cx1_out.at[pl.ds(wid * _M, _M)])
    pltpu.sync_copy(cy2_v, cy2_out.at[pl.ds(wid * _M, _M)])
    pltpu.sync_copy(cx2_v, cx2_out.at[pl.ds(wid * _M, _M)])
    pltpu.sync_copy(thr_v, thr_out.at[pl.ds(wid * 16, 16)])

  return _sc_sel


_sc_select_cached = None


def _sc_select(*args):
    global _sc_select_cached
    if _sc_select_cached is None:
        _sc_select_cached = _build_sc_select()
    return _sc_select_cached(*args)


def _scal_iou_ge(by1a, bx1a, by2a, bx2a, bara, by1b, bx1b, by2b, bx2b, barb):
    yy1 = jnp.maximum(by1a, by1b)
    xx1 = jnp.maximum(bx1a, bx1b)
    yy2 = jnp.minimum(by2a, by2b)
    xx2 = jnp.minimum(bx2a, bx2b)
    inter = jnp.maximum(yy2 - yy1, 0.0) * jnp.maximum(xx2 - xx1, 0.0)
    union = bara + barb - inter
    iou = jnp.where(union > 0.0, inter / union, 0.0)
    return iou >= 0.7


def _nms_cand_body(mos_ref, b0_ref, thr_ref, s_ref, idxf_ref,
                   y1_ref, x1_ref, y2_ref, x2_ref,
                   out_ref, flag_ref, sw_ref, ar_ref, acc_ref):
    sw_ref[...] = s_ref[...]
    ar_ref[...] = (jnp.maximum(y2_ref[...] - y1_ref[...], 0.0)
                   * jnp.maximum(x2_ref[...] - x1_ref[...], 0.0))
    flag_ref[0] = 0
    mos = mos_ref[0]
    mthr = jnp.max(thr_ref[...])
    b0y1 = b0_ref[0]
    b0x1 = b0_ref[1]
    b0y2 = b0_ref[2]
    b0x2 = b0_ref[3]
    out_ref[:, pl.ds(0, 1)] = jnp.full((_MAXO, 1), b0y1)
    out_ref[:, pl.ds(1, 1)] = jnp.full((_MAXO, 1), b0x1)
    out_ref[:, pl.ds(2, 1)] = jnp.full((_MAXO, 1), b0y2)
    out_ref[:, pl.ds(3, 1)] = jnp.full((_MAXO, 1), b0x2)

    neg = jnp.float32(_NEG)
    big = jnp.float32(_NP)

    def body(i, rows):
        acc_ref[0] = 0

        @pl.when(rows < mos)
        def _():
            s = sw_ref[...]
            idxp = idxf_ref[...]
            y1 = y1_ref[...]
            x1 = x1_ref[...]
            y2 = y2_ref[...]
            x2 = x2_ref[...]
            ar = ar_ref[...]

            m1 = jnp.max(s)
            q1 = s == m1
            s2 = jnp.where(q1, neg, s)
            m2 = jnp.max(s2)
            q2 = s2 == m2
            s3 = jnp.where(q2, neg, s2)
            m3 = jnp.max(s3)
            q3 = s3 == m3
            s4 = jnp.where(q3, neg, s3)
            m4 = jnp.max(s4)
            q4 = s4 == m4

            idx1 = jnp.min(jnp.where(q1, idxp, big))
            sel1 = idxp == idx1
            ones = jnp.float32(1.0)
            cnt1 = jnp.sum(jnp.where(q1, ones, 0.0))
            cnt2 = jnp.sum(jnp.where(q2, ones, 0.0))
            cnt3 = jnp.sum(jnp.where(q3, ones, 0.0))
            cnt4 = jnp.sum(jnp.where(q4, ones, 0.0))

            def extract(sel):
                return (jnp.max(jnp.where(sel, y1, neg)),
                        jnp.max(jnp.where(sel, x1, neg)),
                        jnp.max(jnp.where(sel, y2, neg)),
                        jnp.max(jnp.where(sel, x2, neg)),
                        jnp.max(jnp.where(sel, ar, neg)))

            c1 = extract(sel1)
            c2 = extract(q2)
            c3 = extract(q3)
            c4 = extract(q4)

            ok2 = (cnt1 <= 1.0) & (cnt2 <= 1.0) & (m2 > _NEG / 2.0)
            ok3 = ok2 & (cnt3 <= 1.0) & (m3 > _NEG / 2.0)
            ok4 = ok3 & (cnt4 <= 1.0) & (m4 > _NEG / 2.0)


            a2 = ok2 & ~_scal_iou_ge(*c1, *c2)
            a3 = ok3 & ~((a2 & _scal_iou_ge(*c2, *c3))
                         | _scal_iou_ge(*c1, *c3))
            a4 = ok4 & ~((a3 & _scal_iou_ge(*c3, *c4))
                         | (a2 & _scal_iou_ge(*c2, *c4))
                         | _scal_iou_ge(*c1, *c4))

            r1 = rows
            r2 = r1 + 1
            r3 = r2 + jnp.where(a2, 1, 0)
            r4 = r3 + jnp.where(a3, 1, 0)
            w1 = r1 < mos
            w2 = a2 & (r2 < mos)
            w3 = a3 & (r3 < mos)
            w4 = a4 & (r4 < mos)

            mlast = jnp.where(w4, m4, jnp.where(w3, m3,
                              jnp.where(w2, m2, m1)))
            flag_ref[0] = flag_ref[0] | (mlast <= mthr).astype(jnp.int32) \
                | (m1 <= mthr).astype(jnp.int32)

            def emit(w, r, c):
                @pl.when(w)
                def _():
                    out_ref[pl.ds(r, 1), pl.ds(0, 1)] = jnp.full((1, 1), c[0])
                    out_ref[pl.ds(r, 1), pl.ds(1, 1)] = jnp.full((1, 1), c[1])
                    out_ref[pl.ds(r, 1), pl.ds(2, 1)] = jnp.full((1, 1), c[2])
                    out_ref[pl.ds(r, 1), pl.ds(3, 1)] = jnp.full((1, 1), c[3])

            emit(w1, r1, c1)
            emit(w2, r2, c2)
            emit(w3, r3, c3)
            emit(w4, r4, c4)

            def supp_mask(c, selq):
                yy1 = jnp.maximum(c[0], y1)
                xx1 = jnp.maximum(c[1], x1)
                yy2 = jnp.minimum(c[2], y2)
                xx2 = jnp.minimum(c[3], x2)
                inter = (jnp.maximum(yy2 - yy1, 0.0)
                         * jnp.maximum(xx2 - xx1, 0.0))
                union = c[4] + ar - inter
                iou = jnp.where(union > 0.0, inter / union, 0.0)
                return (iou >= 0.7) | selq

            sm = supp_mask(c1, sel1)
            sm = sm | (w2 & supp_mask(c2, q2))
            sm = sm | (w3 & supp_mask(c3, q3))
            sm = sm | (w4 & supp_mask(c4, q4))
            sw_ref[...] = jnp.where(sm, neg, s)

            acc_ref[0] = (jnp.where(w1, 1, 0) + jnp.where(w2, 1, 0)
                          + jnp.where(w3, 1, 0) + jnp.where(w4, 1, 0))

        return rows + acc_ref[0]

    lax.fori_loop(0, _MAXO, body, jnp.int32(0))


def _nms_full_body(mos_ref, s_ref, y1_ref, x1_ref, y2_ref, x2_ref,
                   out_ref, sw_ref, ar_ref):
    sw_ref[...] = s_ref[...]
    ar_ref[...] = (jnp.maximum(y2_ref[...] - y1_ref[...], 0.0)
                   * jnp.maximum(x2_ref[...] - x1_ref[...], 0.0))
    mos = mos_ref[0]
    rows = lax.broadcasted_iota(jnp.int32, (_R, _C), 0)
    cols = lax.broadcasted_iota(jnp.int32, (_R, _C), 1)
    idxg = (rows * _C + cols).astype(jnp.float32)

    b0y1 = y1_ref[0, 0]
    b0x1 = x1_ref[0, 0]
    b0y2 = y2_ref[0, 0]
    b0x2 = x2_ref[0, 0]

    def body(i, _):
        s = sw_ref[...]
        m = jnp.max(s)
        idx = jnp.min(jnp.where(s == m, idxg, jnp.float32(_NP)))
        valid = (m > _NEG / 2.0) & (i < mos)
        sel = idxg == idx
        by1 = jnp.max(jnp.where(sel, y1_ref[...], _NEG))
        bx1 = jnp.max(jnp.where(sel, x1_ref[...], _NEG))
        by2 = jnp.max(jnp.where(sel, y2_ref[...], _NEG))
        bx2 = jnp.max(jnp.where(sel, x2_ref[...], _NEG))
        barea = jnp.max(jnp.where(sel, ar_ref[...], _NEG))

        yy1 = jnp.maximum(by1, y1_ref[...])
        xx1 = jnp.maximum(bx1, x1_ref[...])
        yy2 = jnp.minimum(by2, y2_ref[...])
        xx2 = jnp.minimum(bx2, x2_ref[...])
        inter = jnp.maximum(yy2 - yy1, 0.0) * jnp.maximum(xx2 - xx1, 0.0)
        union = barea + ar_ref[...] - inter
        iou = jnp.where(union > 0.0, inter / union, 0.0)
        supp = (iou >= 0.7) | sel
        sw_ref[...] = jnp.where(supp, _NEG, s)

        oy1 = jnp.where(valid, by1, b0y1)
        ox1 = jnp.where(valid, bx1, b0x1)
        oy2 = jnp.where(valid, by2, b0y2)
        ox2 = jnp.where(valid, bx2, b0x2)
        out_ref[pl.ds(i, 1), pl.ds(0, 1)] = jnp.full((1, 1), oy1)
        out_ref[pl.ds(i, 1), pl.ds(1, 1)] = jnp.full((1, 1), ox1)
        out_ref[pl.ds(i, 1), pl.ds(2, 1)] = jnp.full((1, 1), oy2)
        out_ref[pl.ds(i, 1), pl.ds(3, 1)] = jnp.full((1, 1), ox2)
        return 0

    lax.fori_loop(0, _MAXO, body, 0)


def kernel(rois, scores, max_output_size):
    s = jnp.squeeze(scores, axis=-1)
    s_p = jnp.concatenate([s, jnp.full((_NP - _N,), _NEG, jnp.float32)])
    zpad = jnp.zeros((_NP - _N,), jnp.float32)
    y1 = jnp.concatenate([rois[:, 0], zpad])
    x1 = jnp.concatenate([rois[:, 1], zpad])
    y2 = jnp.concatenate([rois[:, 2], zpad])
    x2 = jnp.concatenate([rois[:, 3], zpad])
    mos = jnp.asarray(max_output_size, jnp.int32).reshape(1)
    b0 = rois[0]

    cs, ci, cy1, cx1, cy2, cx2, thr = _sc_select(s_p, y1, x1, y2, x2)

    vspec = pl.BlockSpec(memory_space=pltpu.VMEM)
    sspec = pl.BlockSpec(memory_space=pltpu.SMEM)
    fast_out, flag = pl.pallas_call(
        _nms_cand_body,
        out_shape=[jax.ShapeDtypeStruct((_MAXO, 4), jnp.float32),
                   jax.ShapeDtypeStruct((1,), jnp.int32)],
        in_specs=[sspec, sspec] + [vspec] * 7,
        out_specs=[vspec, sspec],
        scratch_shapes=[pltpu.VMEM((_CR, _CC), jnp.float32),
                        pltpu.VMEM((_CR, _CC), jnp.float32),
                        pltpu.SMEM((1,), jnp.int32)],
    )(mos, b0, thr.reshape(8, -1), cs.reshape(_CR, _CC),
      ci.astype(jnp.float32).reshape(_CR, _CC), cy1.reshape(_CR, _CC),
      cx1.reshape(_CR, _CC), cy2.reshape(_CR, _CC), cx2.reshape(_CR, _CC))

    def _full(_):
        return pl.pallas_call(
            _nms_full_body,
            out_shape=jax.ShapeDtypeStruct((_MAXO, 4), jnp.float32),
            in_specs=[sspec] + [vspec] * 5,
            scratch_shapes=[pltpu.VMEM((_R, _C), jnp.float32),
                            pltpu.VMEM((_R, _C), jnp.float32)],
        )(mos, s_p.reshape(_R, _C), y1.reshape(_R, _C), x1.reshape(_R, _C),
          y2.reshape(_R, _C), x2.reshape(_R, _C))

    def _fast(_):
        return fast_out

    return lax.cond(flag[0] > 0, _full, _fast, None)


# speedup vs baseline: 67.3132x; 1.0506x over previous
import functools

import numpy as np

import jax
import jax.numpy as jnp
from jax import lax
from jax.experimental import pallas as pl
from jax.experimental.pallas import tpu as pltpu
from jax.experimental.pallas import tpu_sc as plsc

_N = 20000
_NP = 20480
_NW = 32
_PT = _NP // _NW
_NG = _PT // 64
_M = _NG * 16
_NC = _NW * _M
_CR, _CC = 8, _NC // 8
_R, _C = 8, 2560
_MAXO = 1000
_NEG = float(np.float32(-1e30))


def _lexgt(s, i, sp, ip):
    return (s > sp) | ((s == sp) & (i < ip))


def _ce_stage(lane, s, idx, j, want_max):
    perm = lane ^ j
    sp = s[perm]
    ip = idx[perm]
    gt = _lexgt(s, idx, sp, ip)
    gi = jnp.where(gt, 1, 0)
    wi = jnp.where(want_max, 1, 0)
    take_self = (gi ^ wi) == 0
    return jnp.where(take_self, s, sp), jnp.where(take_self, idx, ip)


def _sort16_desc(lane, s, idx):
    for k in (2, 4, 8, 16):
        j = k // 2
        while j >= 1:
            jb = (lane >> j.bit_length() - 1) & 1
            kb = (lane >> k.bit_length() - 1) & 1
            want_max = jb == kb
            s, idx = _ce_stage(lane, s, idx, j, want_max)
            j //= 2
    return s, idx


def _merge_top16(lane, sa, ia, sb, ib):
    rperm = lane ^ 15
    rb = sb[rperm]
    rib = ib[rperm]
    gt = _lexgt(sa, ia, rb, rib)
    ts = jnp.where(gt, sa, rb)
    ti = jnp.where(gt, ia, rib)
    bs = jnp.where(gt, rb, sa)
    j = 8
    while j >= 1:
        want_max = (lane & j) == 0
        ts, ti = _ce_stage(lane, ts, ti, j, want_max)
        j //= 2
    bmax = bs
    for d in (8, 4, 2, 1):
        bmax = jnp.maximum(bmax, bmax[lane ^ d])
    return ts, ti, bmax


def _build_sc_select():
  mesh = plsc.VectorSubcoreMesh(core_axis_name="c", subcore_axis_name="s")

  @functools.partial(
    pl.kernel,
    mesh=mesh,
    out_type=[
        jax.ShapeDtypeStruct((_NC,), jnp.float32),
        jax.ShapeDtypeStruct((_NC,), jnp.int32),
        jax.ShapeDtypeStruct((_NC,), jnp.float32),
        jax.ShapeDtypeStruct((_NC,), jnp.float32),
        jax.ShapeDtypeStruct((_NC,), jnp.float32),
        jax.ShapeDtypeStruct((_NC,), jnp.float32),
        jax.ShapeDtypeStruct((_NW * 16,), jnp.float32),
    ],
    scratch_types=[
        pltpu.VMEM((_PT,), jnp.float32),
        pltpu.VMEM((_PT,), jnp.float32),
        pltpu.VMEM((_PT,), jnp.float32),
        pltpu.VMEM((_PT,), jnp.float32),
        pltpu.VMEM((_PT,), jnp.float32),
        pltpu.VMEM((_M,), jnp.float32),
        pltpu.VMEM((_M,), jnp.int32),
        pltpu.VMEM((_M,), jnp.float32),
        pltpu.VMEM((_M,), jnp.float32),
        pltpu.VMEM((_M,), jnp.float32),
        pltpu.VMEM((_M,), jnp.float32),
        pltpu.VMEM((16,), jnp.float32),
    ],
  )
  def _sc_sel(s_hbm, y1_hbm, x1_hbm, y2_hbm, x2_hbm,
              cs_out, ci_out, cy1_out, cx1_out, cy2_out, cx2_out, thr_out,
              s_v, y1_v, x1_v, y2_v, x2_v,
              cs_v, ci_v, cy1_v, cx1_v, cy2_v, cx2_v, thr_v):
    wid = lax.axis_index("s") * 2 + lax.axis_index("c")
    base = wid * _PT
    pltpu.sync_copy(s_hbm.at[pl.ds(base, _PT)], s_v)
    pltpu.sync_copy(y1_hbm.at[pl.ds(base, _PT)], y1_v)
    pltpu.sync_copy(x1_hbm.at[pl.ds(base, _PT)], x1_v)
    pltpu.sync_copy(y2_hbm.at[pl.ds(base, _PT)], y2_v)
    pltpu.sync_copy(x2_hbm.at[pl.ds(base, _PT)], x2_v)

    lane = lax.iota(jnp.int32, 16)
    neg = jnp.float32(_NEG)

    def _group(g, thrmax):
        goff = g * 64
        vs, vi = [], []
        for k in range(4):
            sv = s_v[pl.ds(goff + k * 16, 16)]
            gi = base + goff + k * 16 + lane
            ss, si = _sort16_desc(lane, sv, gi)
            vs.append(ss)
            vi.append(si)
        t01s, t01i, b01 = _merge_top16(lane, vs[0], vi[0], vs[1], vi[1])
        t23s, t23i, b23 = _merge_top16(lane, vs[2], vi[2], vs[3], vi[3])
        ts, ti, bt = _merge_top16(lane, t01s, t01i, t23s, t23i)
        thr_g = jnp.maximum(jnp.maximum(b01, b23), bt)
        thrmax = jnp.maximum(thrmax, thr_g)

        li = ti - (base + goff)
        vno = li >> 4
        lno = li - (vno << 4)
        outs = []
        for c_v in (y1_v, x1_v, y2_v, x2_v):
            r = jnp.zeros((16,), jnp.float32)
            for k in range(4):
                ck = c_v[pl.ds(goff + k * 16, 16)]
                r = jnp.where(vno == k, ck[lno], r)
            outs.append(r)

        cs_v[pl.ds(g * 16, 16)] = ts
        ci_v[pl.ds(g * 16, 16)] = ti
        cy1_v[pl.ds(g * 16, 16)] = outs[0]
        cx1_v[pl.ds(g * 16, 16)] = outs[1]
        cy2_v[pl.ds(g * 16, 16)] = outs[2]
        cx2_v[pl.ds(g * 16, 16)] = outs[3]
        return thrmax

    thrmax = lax.fori_loop(0, _NG, _group, jnp.full((16,), neg))
    thr_v[...] = thrmax

    pltpu.sync_copy(cs_v, cs_out.at[pl.ds(wid * _M, _M)])
    pltpu.sync_copy(ci_v, ci_out.at[pl.ds(wid * _M, _M)])
    pltpu.sync_copy(cy1_v, cy1_out.at[pl.ds(wid * _M, _M)])
    pltpu.sync_copy(cx1_v, cx1_out.at[pl.ds(wid * _M, _M)])
    pltpu.sync_copy(cy2_v, cy2_out.at[pl.ds(wid * _M, _M)])
    pltpu.sync_copy(cx2_v, cx2_out.at[pl.ds(wid * _M, _M)])
    pltpu.sync_copy(thr_v, thr_out.at[pl.ds(wid * 16, 16)])

  return _sc_sel


_sc_select_cached = None


def _sc_select(*args):
    global _sc_select_cached
    if _sc_select_cached is None:
        _sc_select_cached = _build_sc_select()
    return _sc_select_cached(*args)


def _scal_iou_ge(by1a, bx1a, by2a, bx2a, bara, by1b, bx1b, by2b, bx2b, barb):
    yy1 = jnp.maximum(by1a, by1b)
    xx1 = jnp.maximum(bx1a, bx1b)
    yy2 = jnp.minimum(by2a, by2b)
    xx2 = jnp.minimum(bx2a, bx2b)
    inter = jnp.maximum(yy2 - yy1, 0.0) * jnp.maximum(xx2 - xx1, 0.0)
    union = bara + barb - inter
    iou = jnp.where(union > 0.0, inter / union, 0.0)
    return iou >= 0.7


def _nms_cand_body(mos_ref, b0_ref, thr_ref, s_ref, idxf_ref,
                   y1_ref, x1_ref, y2_ref, x2_ref,
                   out_ref, flag_ref, sw_ref, ar_ref, acc_ref):
    sw_ref[...] = s_ref[...]
    ar_ref[...] = (jnp.maximum(y2_ref[...] - y1_ref[...], 0.0)
                   * jnp.maximum(x2_ref[...] - x1_ref[...], 0.0))
    flag_ref[0] = 0
    mos = mos_ref[0]
    mthr = jnp.max(thr_ref[...])
    b0y1 = b0_ref[0]
    b0x1 = b0_ref[1]
    b0y2 = b0_ref[2]
    b0x2 = b0_ref[3]
    out_ref[:, pl.ds(0, 1)] = jnp.full((_MAXO, 1), b0y1)
    out_ref[:, pl.ds(1, 1)] = jnp.full((_MAXO, 1), b0x1)
    out_ref[:, pl.ds(2, 1)] = jnp.full((_MAXO, 1), b0y2)
    out_ref[:, pl.ds(3, 1)] = jnp.full((_MAXO, 1), b0x2)

    neg = jnp.float32(_NEG)
    big = jnp.float32(_NP)

    def body(i, rows):
        acc_ref[0] = 0

        @pl.when(rows < mos)
        def _():
            s = sw_ref[...]
            idxp = idxf_ref[...]
            y1 = y1_ref[...]
            x1 = x1_ref[...]
            y2 = y2_ref[...]
            x2 = x2_ref[...]
            ar = ar_ref[...]

            m1 = jnp.max(s)
            q1 = s == m1
            s2 = jnp.where(q1, neg, s)
            m2 = jnp.max(s2)
            q2 = s2 == m2
            s3 = jnp.where(q2, neg, s2)
            m3 = jnp.max(s3)
            q3 = s3 == m3
            s4 = jnp.where(q3, neg, s3)
            m4 = jnp.max(s4)
            q4 = s4 == m4

            idx1 = jnp.min(jnp.where(q1, idxp, big))
            sel1 = idxp == idx1
            ones = jnp.float32(1.0)
            cnt1 = jnp.sum(jnp.where(q1, ones, 0.0))
            cnt2 = jnp.sum(jnp.where(q2, ones, 0.0))
            cnt3 = jnp.sum(jnp.where(q3, ones, 0.0))
            cnt4 = jnp.sum(jnp.where(q4, ones, 0.0))

            def extract(sel):
                return (jnp.max(jnp.where(sel, y1, neg)),
                        jnp.max(jnp.where(sel, x1, neg)),
                        jnp.max(jnp.where(sel, y2, neg)),
                        jnp.max(jnp.where(sel, x2, neg)),
                        jnp.max(jnp.where(sel, ar, neg)))

            c1 = extract(sel1)
            c2 = extract(q2)
            c3 = extract(q3)
            c4 = extract(q4)

            ok2 = (cnt1 <= 1.0) & (cnt2 <= 1.0) & (m2 > _NEG / 2.0)
            ok3 = ok2 & (cnt3 <= 1.0) & (m3 > _NEG / 2.0)
            ok4 = ok3 & (cnt4 <= 1.0) & (m4 > _NEG / 2.0)


            a2 = ok2 & ~_scal_iou_ge(*c1, *c2)
            a3 = ok3 & ~((a2 & _scal_iou_ge(*c2, *c3))
                         | _scal_iou_ge(*c1, *c3))
            a4 = ok4 & ~((a3 & _scal_iou_ge(*c3, *c4))
                         | (a2 & _scal_iou_ge(*c2, *c4))
                         | _scal_iou_ge(*c1, *c4))

            r1 = rows
            r2 = r1 + 1
            r3 = r2 + jnp.where(a2, 1, 0)
            r4 = r3 + jnp.where(a3, 1, 0)
            w1 = r1 < mos
            w2 = a2 & (r2 < mos)
            w3 = a3 & (r3 < mos)
            w4 = a4 & (r4 < mos)

            mlast = jnp.where(w4, m4, jnp.where(w3, m3,
                              jnp.where(w2, m2, m1)))
            flag_ref[0] = flag_ref[0] | (mlast <= mthr).astype(jnp.int32) \
                | (m1 <= mthr).astype(jnp.int32)

            def emit(w, r, c):
                @pl.when(w)
                def _():
                    out_ref[pl.ds(r, 1), pl.ds(0, 1)] = jnp.full((1, 1), c[0])
                    out_ref[pl.ds(r, 1), pl.ds(1, 1)] = jnp.full((1, 1), c[1])
                    out_ref[pl.ds(r, 1), pl.ds(2, 1)] = jnp.full((1, 1), c[2])
                    out_ref[pl.ds(r, 1), pl.ds(3, 1)] = jnp.full((1, 1), c[3])

            emit(w1, r1, c1)
            emit(w2, r2, c2)
            emit(w3, r3, c3)
            emit(w4, r4, c4)

            def supp_mask(c, selq):
                yy1 = jnp.maximum(c[0], y1)
                xx1 = jnp.maximum(c[1], x1)
                yy2 = jnp.minimum(c[2], y2)
                xx2 = jnp.minimum(c[3], x2)
                inter = (jnp.maximum(yy2 - yy1, 0.0)
                         * jnp.maximum(xx2 - xx1, 0.0))
                union = c[4] + ar - inter
                iou = jnp.where(union > 0.0, inter / union, 0.0)
                return (iou >= 0.7) | selq

            sm = supp_mask(c1, sel1)
            sm = sm | (w2 & supp_mask(c2, q2))
            sm = sm | (w3 & supp_mask(c3, q3))
            sm = sm | (w4 & supp_mask(c4, q4))
            sw_ref[...] = jnp.where(sm, neg, s)

            acc_ref[0] = (jnp.where(w1, 1, 0) + jnp.where(w2, 1, 0)
                          + jnp.where(w3, 1, 0) + jnp.where(w4, 1, 0))

        return rows + acc_ref[0]

    rows = lax.fori_loop(0, 320, body, jnp.int32(0))
    flag_ref[0] = flag_ref[0] | (rows < mos).astype(jnp.int32)


def _nms_full_body(mos_ref, s_ref, y1_ref, x1_ref, y2_ref, x2_ref,
                   out_ref, sw_ref, ar_ref):
    sw_ref[...] = s_ref[...]
    ar_ref[...] = (jnp.maximum(y2_ref[...] - y1_ref[...], 0.0)
                   * jnp.maximum(x2_ref[...] - x1_ref[...], 0.0))
    mos = mos_ref[0]
    rows = lax.broadcasted_iota(jnp.int32, (_R, _C), 0)
    cols = lax.broadcasted_iota(jnp.int32, (_R, _C), 1)
    idxg = (rows * _C + cols).astype(jnp.float32)

    b0y1 = y1_ref[0, 0]
    b0x1 = x1_ref[0, 0]
    b0y2 = y2_ref[0, 0]
    b0x2 = x2_ref[0, 0]

    def body(i, _):
        s = sw_ref[...]
        m = jnp.max(s)
        idx = jnp.min(jnp.where(s == m, idxg, jnp.float32(_NP)))
        valid = (m > _NEG / 2.0) & (i < mos)
        sel = idxg == idx
        by1 = jnp.max(jnp.where(sel, y1_ref[...], _NEG))
        bx1 = jnp.max(jnp.where(sel, x1_ref[...], _NEG))
        by2 = jnp.max(jnp.where(sel, y2_ref[...], _NEG))
        bx2 = jnp.max(jnp.where(sel, x2_ref[...], _NEG))
        barea = jnp.max(jnp.where(sel, ar_ref[...], _NEG))

        yy1 = jnp.maximum(by1, y1_ref[...])
        xx1 = jnp.maximum(bx1, x1_ref[...])
        yy2 = jnp.minimum(by2, y2_ref[...])
        xx2 = jnp.minimum(bx2, x2_ref[...])
        inter = jnp.maximum(yy2 - yy1, 0.0) * jnp.maximum(xx2 - xx1, 0.0)
        union = barea + ar_ref[...] - inter
        iou = jnp.where(union > 0.0, inter / union, 0.0)
        supp = (iou >= 0.7) | sel
        sw_ref[...] = jnp.where(supp, _NEG, s)

        oy1 = jnp.where(valid, by1, b0y1)
        ox1 = jnp.where(valid, bx1, b0x1)
        oy2 = jnp.where(valid, by2, b0y2)
        ox2 = jnp.where(valid, bx2, b0x2)
        out_ref[pl.ds(i, 1), pl.ds(0, 1)] = jnp.full((1, 1), oy1)
        out_ref[pl.ds(i, 1), pl.ds(1, 1)] = jnp.full((1, 1), ox1)
        out_ref[pl.ds(i, 1), pl.ds(2, 1)] = jnp.full((1, 1), oy2)
        out_ref[pl.ds(i, 1), pl.ds(3, 1)] = jnp.full((1, 1), ox2)
        return 0

    lax.fori_loop(0, _MAXO, body, 0)


def kernel(rois, scores, max_output_size):
    s = jnp.squeeze(scores, axis=-1)
    s_p = jnp.concatenate([s, jnp.full((_NP - _N,), _NEG, jnp.float32)])
    zpad = jnp.zeros((_NP - _N,), jnp.float32)
    y1 = jnp.concatenate([rois[:, 0], zpad])
    x1 = jnp.concatenate([rois[:, 1], zpad])
    y2 = jnp.concatenate([rois[:, 2], zpad])
    x2 = jnp.concatenate([rois[:, 3], zpad])
    mos = jnp.asarray(max_output_size, jnp.int32).reshape(1)
    b0 = rois[0]

    cs, ci, cy1, cx1, cy2, cx2, thr = _sc_select(s_p, y1, x1, y2, x2)

    vspec = pl.BlockSpec(memory_space=pltpu.VMEM)
    sspec = pl.BlockSpec(memory_space=pltpu.SMEM)
    fast_out, flag = pl.pallas_call(
        _nms_cand_body,
        out_shape=[jax.ShapeDtypeStruct((_MAXO, 4), jnp.float32),
                   jax.ShapeDtypeStruct((1,), jnp.int32)],
        in_specs=[sspec, sspec] + [vspec] * 7,
        out_specs=[vspec, sspec],
        scratch_shapes=[pltpu.VMEM((_CR, _CC), jnp.float32),
                        pltpu.VMEM((_CR, _CC), jnp.float32),
                        pltpu.SMEM((1,), jnp.int32)],
    )(mos, b0, thr.reshape(8, -1), cs.reshape(_CR, _CC),
      ci.astype(jnp.float32).reshape(_CR, _CC), cy1.reshape(_CR, _CC),
      cx1.reshape(_CR, _CC), cy2.reshape(_CR, _CC), cx2.reshape(_CR, _CC))

    def _full(_):
        return pl.pallas_call(
            _nms_full_body,
            out_shape=jax.ShapeDtypeStruct((_MAXO, 4), jnp.float32),
            in_specs=[sspec] + [vspec] * 5,
            scratch_shapes=[pltpu.VMEM((_R, _C), jnp.float32),
                            pltpu.VMEM((_R, _C), jnp.float32)],
        )(mos, s_p.reshape(_R, _C), y1.reshape(_R, _C), x1.reshape(_R, _C),
          y2.reshape(_R, _C), x2.reshape(_R, _C))

    def _fast(_):
        return fast_out

    return lax.cond(flag[0] > 0, _full, _fast, None)


# speedup vs baseline: 72.0425x; 1.0703x over previous
import functools

import numpy as np

import jax
import jax.numpy as jnp
from jax import lax
from jax.experimental import pallas as pl
from jax.experimental.pallas import tpu as pltpu
from jax.experimental.pallas import tpu_sc as plsc

_N = 20000
_NP = 20480
_NW = 32
_PT = _NP // _NW
_NG = _PT // 64
_M = _NG * 16
_NC = _NW * _M
_CR, _CC = 8, _NC // 8
_R, _C = 8, 2560
_MAXO = 1000
_NEG = float(np.float32(-1e30))


def _lexgt(s, i, sp, ip):
    return (s > sp) | ((s == sp) & (i < ip))


def _ce_stage(lane, s, idx, j, want_max):
    perm = lane ^ j
    sp = s[perm]
    ip = idx[perm]
    gt = _lexgt(s, idx, sp, ip)
    gi = jnp.where(gt, 1, 0)
    wi = jnp.where(want_max, 1, 0)
    take_self = (gi ^ wi) == 0
    return jnp.where(take_self, s, sp), jnp.where(take_self, idx, ip)


def _sort16_desc(lane, s, idx):
    for k in (2, 4, 8, 16):
        j = k // 2
        while j >= 1:
            jb = (lane >> j.bit_length() - 1) & 1
            kb = (lane >> k.bit_length() - 1) & 1
            want_max = jb == kb
            s, idx = _ce_stage(lane, s, idx, j, want_max)
            j //= 2
    return s, idx


def _merge_top16(lane, sa, ia, sb, ib):
    rperm = lane ^ 15
    rb = sb[rperm]
    rib = ib[rperm]
    gt = _lexgt(sa, ia, rb, rib)
    ts = jnp.where(gt, sa, rb)
    ti = jnp.where(gt, ia, rib)
    bs = jnp.where(gt, rb, sa)
    j = 8
    while j >= 1:
        want_max = (lane & j) == 0
        ts, ti = _ce_stage(lane, ts, ti, j, want_max)
        j //= 2
    bmax = bs
    for d in (8, 4, 2, 1):
        bmax = jnp.maximum(bmax, bmax[lane ^ d])
    return ts, ti, bmax


def _build_sc_select():
  mesh = plsc.VectorSubcoreMesh(core_axis_name="c", subcore_axis_name="s")

  @functools.partial(
    pl.kernel,
    mesh=mesh,
    out_type=[
        jax.ShapeDtypeStruct((_NC,), jnp.float32),
        jax.ShapeDtypeStruct((_NC,), jnp.int32),
        jax.ShapeDtypeStruct((_NC,), jnp.float32),
        jax.ShapeDtypeStruct((_NC,), jnp.float32),
        jax.ShapeDtypeStruct((_NC,), jnp.float32),
        jax.ShapeDtypeStruct((_NC,), jnp.float32),
        jax.ShapeDtypeStruct((_NW * 16,), jnp.float32),
    ],
    scratch_types=[
        pltpu.VMEM((_PT,), jnp.float32),
        pltpu.VMEM((_PT,), jnp.float32),
        pltpu.VMEM((_PT,), jnp.float32),
        pltpu.VMEM((_PT,), jnp.float32),
        pltpu.VMEM((_PT,), jnp.float32),
        pltpu.VMEM((_M,), jnp.float32),
        pltpu.VMEM((_M,), jnp.int32),
        pltpu.VMEM((_M,), jnp.float32),
        pltpu.VMEM((_M,), jnp.float32),
        pltpu.VMEM((_M,), jnp.float32),
        pltpu.VMEM((_M,), jnp.float32),
        pltpu.VMEM((16,), jnp.float32),
    ],
  )
  def _sc_sel(s_hbm, y1_hbm, x1_hbm, y2_hbm, x2_hbm,
              cs_out, ci_out, cy1_out, cx1_out, cy2_out, cx2_out, thr_out,
              s_v, y1_v, x1_v, y2_v, x2_v,
              cs_v, ci_v, cy1_v, cx1_v, cy2_v, cx2_v, thr_v):
    wid = lax.axis_index("s") * 2 + lax.axis_index("c")
    base = wid * _PT
    pltpu.sync_copy(s_hbm.at[pl.ds(base, _PT)], s_v)
    pltpu.sync_copy(y1_hbm.at[pl.ds(base, _PT)], y1_v)
    pltpu.sync_copy(x1_hbm.at[pl.ds(base, _PT)], x1_v)
    pltpu.sync_copy(y2_hbm.at[pl.ds(base, _PT)], y2_v)
    pltpu.sync_copy(x2_hbm.at[pl.ds(base, _PT)], x2_v)

    lane = lax.iota(jnp.int32, 16)
    neg = jnp.float32(_NEG)

    def _group(g, thrmax):
        goff = g * 64
        vs, vi = [], []
        for k in range(4):
            sv = s_v[pl.ds(goff + k * 16, 16)]
            gi = base + goff + k * 16 + lane
            ss, si = _sort16_desc(lane, sv, gi)
            vs.append(ss)
            vi.append(si)
        t01s, t01i, b01 = _merge_top16(lane, vs[0], vi[0], vs[1], vi[1])
        t23s, t23i, b23 = _merge_top16(lane, vs[2], vi[2], vs[3], vi[3])
        ts, ti, bt = _merge_top16(lane, t01s, t01i, t23s, t23i)
        thr_g = jnp.maximum(jnp.maximum(b01, b23), bt)
        thrmax = jnp.maximum(thrmax, thr_g)

        li = ti - (base + goff)
        vno = li >> 4
        lno = li - (vno << 4)
        outs = []
        for c_v in (y1_v, x1_v, y2_v, x2_v):
            r = jnp.zeros((16,), jnp.float32)
            for k in range(4):
                ck = c_v[pl.ds(goff + k * 16, 16)]
                r = jnp.where(vno == k, ck[lno], r)
            outs.append(r)

        cs_v[pl.ds(g * 16, 16)] = ts
        ci_v[pl.ds(g * 16, 16)] = ti
        cy1_v[pl.ds(g * 16, 16)] = outs[0]
        cx1_v[pl.ds(g * 16, 16)] = outs[1]
        cy2_v[pl.ds(g * 16, 16)] = outs[2]
        cx2_v[pl.ds(g * 16, 16)] = outs[3]
        return thrmax

    thrmax = lax.fori_loop(0, _NG, _group, jnp.full((16,), neg))
    thr_v[...] = thrmax

    pltpu.sync_copy(cs_v, cs_out.at[pl.ds(wid * _M, _M)])
    pltpu.sync_copy(ci_v, ci_out.at[pl.ds(wid * _M, _M)])
    pltpu.sync_copy(cy1_v, cy1_out.at[pl.ds(wid * _M, _M)])
    pltpu.sync_copy(cx1_v, cx1_out.at[pl.ds(wid * _M, _M)])
    pltpu.sync_copy(cy2_v, cy2_out.at[pl.ds(wid * _M, _M)])
    pltpu.sync_copy(cx2_v, cx2_out.at[pl.ds(wid * _M, _M)])
    pltpu.sync_copy(thr_v, thr_out.at[pl.ds(wid * 16, 16)])

  return _sc_sel


_sc_select_cached = None


def _sc_select(*args):
    global _sc_select_cached
    if _sc_select_cached is None:
        _sc_select_cached = _build_sc_select()
    return _sc_select_cached(*args)


def _scal_iou_ge(by1a, bx1a, by2a, bx2a, bara, by1b, bx1b, by2b, bx2b, barb):
    yy1 = jnp.maximum(by1a, by1b)
    xx1 = jnp.maximum(bx1a, bx1b)
    yy2 = jnp.minimum(by2a, by2b)
    xx2 = jnp.minimum(bx2a, bx2b)
    inter = jnp.maximum(yy2 - yy1, 0.0) * jnp.maximum(xx2 - xx1, 0.0)
    union = bara + barb - inter
    iou = jnp.where(union > 0.0, inter / union, 0.0)
    return iou >= 0.7


def _nms_cand_body(mos_ref, b0_ref, thr_ref, s_ref, idxf_ref,
                   y1_ref, x1_ref, y2_ref, x2_ref,
                   out_ref, flag_ref, sw_ref, ar_ref, acc_ref):
    sw_ref[...] = s_ref[...]
    ar_ref[...] = (jnp.maximum(y2_ref[...] - y1_ref[...], 0.0)
                   * jnp.maximum(x2_ref[...] - x1_ref[...], 0.0))
    mos = mos_ref[0]
    flag_ref[0] = (mos != _MAXO).astype(jnp.int32)
    mthr = jnp.max(thr_ref[...])
    b0y1 = b0_ref[0]
    b0x1 = b0_ref[1]
    b0y2 = b0_ref[2]
    b0x2 = b0_ref[3]
    out_ref[:, pl.ds(0, 1)] = jnp.full((_MAXO + 8, 1), b0y1)
    out_ref[:, pl.ds(1, 1)] = jnp.full((_MAXO + 8, 1), b0x1)
    out_ref[:, pl.ds(2, 1)] = jnp.full((_MAXO + 8, 1), b0y2)
    out_ref[:, pl.ds(3, 1)] = jnp.full((_MAXO + 8, 1), b0x2)
    row8 = lax.broadcasted_iota(jnp.int32, (8, 4), 0)
    col8 = lax.broadcasted_iota(jnp.int32, (8, 4), 1)

    neg = jnp.float32(_NEG)
    big = jnp.float32(_NP)

    def body(i, rows):
        acc_ref[0] = 0

        @pl.when(rows < mos)
        def _():
            s = sw_ref[...]
            idxp = idxf_ref[...]
            y1 = y1_ref[...]
            x1 = x1_ref[...]
            y2 = y2_ref[...]
            x2 = x2_ref[...]
            ar = ar_ref[...]

            m1 = jnp.max(s)
            q1 = s == m1
            s2 = jnp.where(q1, neg, s)
            m2 = jnp.max(s2)
            q2 = s2 == m2
            s3 = jnp.where(q2, neg, s2)
            m3 = jnp.max(s3)
            q3 = s3 == m3
            s4 = jnp.where(q3, neg, s3)
            m4 = jnp.max(s4)
            q4 = s4 == m4

            idx1 = jnp.min(jnp.where(q1, idxp, big))
            sel1 = idxp == idx1
            ones = jnp.float32(1.0)
            cnt1 = jnp.sum(jnp.where(q1, ones, 0.0))
            cnt2 = jnp.sum(jnp.where(q2, ones, 0.0))
            cnt3 = jnp.sum(jnp.where(q3, ones, 0.0))
            cnt4 = jnp.sum(jnp.where(q4, ones, 0.0))

            def extract(sel):
                return (jnp.max(jnp.where(sel, y1, neg)),
                        jnp.max(jnp.where(sel, x1, neg)),
                        jnp.max(jnp.where(sel, y2, neg)),
                        jnp.max(jnp.where(sel, x2, neg)),
                        jnp.max(jnp.where(sel, ar, neg)))

            c1 = extract(sel1)
            c2 = extract(q2)
            c3 = extract(q3)
            c4 = extract(q4)

            ok2 = (cnt1 <= 1.0) & (cnt2 <= 1.0) & (m2 > _NEG / 2.0)
            ok3 = ok2 & (cnt3 <= 1.0) & (m3 > _NEG / 2.0)
            ok4 = ok3 & (cnt4 <= 1.0) & (m4 > _NEG / 2.0)


            a2 = ok2 & ~_scal_iou_ge(*c1, *c2)
            a3 = ok3 & ~((a2 & _scal_iou_ge(*c2, *c3))
                         | _scal_iou_ge(*c1, *c3))
            a4 = ok4 & ~((a3 & _scal_iou_ge(*c3, *c4))
                         | (a2 & _scal_iou_ge(*c2, *c4))
                         | _scal_iou_ge(*c1, *c4))

            r1 = rows
            r2 = r1 + 1
            r3 = r2 + jnp.where(a2, 1, 0)
            r4 = r3 + jnp.where(a3, 1, 0)
            w1 = r1 < mos
            w2 = a2 & (r2 < mos)
            w3 = a3 & (r3 < mos)
            w4 = a4 & (r4 < mos)

            mlast = jnp.where(w4, m4, jnp.where(w3, m3,
                              jnp.where(w2, m2, m1)))
            flag_ref[0] = flag_ref[0] | (mlast <= mthr).astype(jnp.int32) \
                | (m1 <= mthr).astype(jnp.int32)

            blk = jnp.where(col8 == 0, b0y1,
                  jnp.where(col8 == 1, b0x1,
                  jnp.where(col8 == 2, b0y2, b0x2)))
            for w, r, c in ((w1, r1, c1), (w2, r2, c2),
                            (w3, r3, c3), (w4, r4, c4)):
                hit = w & (row8 == (r - rows))
                blk = jnp.where(hit & (col8 == 0), c[0],
                      jnp.where(hit & (col8 == 1), c[1],
                      jnp.where(hit & (col8 == 2), c[2],
                      jnp.where(hit & (col8 == 3), c[3], blk))))
            out_ref[pl.ds(rows, 8), :] = blk

            def supp_mask(c, selq):
                yy1 = jnp.maximum(c[0], y1)
                xx1 = jnp.maximum(c[1], x1)
                yy2 = jnp.minimum(c[2], y2)
                xx2 = jnp.minimum(c[3], x2)
                inter = (jnp.maximum(yy2 - yy1, 0.0)
                         * jnp.maximum(xx2 - xx1, 0.0))
                union = c[4] + ar - inter
                iou = jnp.where(union > 0.0, inter / union, 0.0)
                return (iou >= 0.7) | selq

            sm = supp_mask(c1, sel1)
            sm = sm | (w2 & supp_mask(c2, q2))
            sm = sm | (w3 & supp_mask(c3, q3))
            sm = sm | (w4 & supp_mask(c4, q4))
            sw_ref[...] = jnp.where(sm, neg, s)

            acc_ref[0] = (jnp.where(w1, 1, 0) + jnp.where(w2, 1, 0)
                          + jnp.where(w3, 1, 0) + jnp.where(w4, 1, 0))

        return rows + acc_ref[0]

    rows = lax.fori_loop(0, 320, body, jnp.int32(0))
    flag_ref[0] = flag_ref[0] | (rows < mos).astype(jnp.int32)


def _nms_full_body(mos_ref, s_ref, y1_ref, x1_ref, y2_ref, x2_ref,
                   out_ref, sw_ref, ar_ref):
    sw_ref[...] = s_ref[...]
    ar_ref[...] = (jnp.maximum(y2_ref[...] - y1_ref[...], 0.0)
                   * jnp.maximum(x2_ref[...] - x1_ref[...], 0.0))
    mos = mos_ref[0]
    rows = lax.broadcasted_iota(jnp.int32, (_R, _C), 0)
    cols = lax.broadcasted_iota(jnp.int32, (_R, _C), 1)
    idxg = (rows * _C + cols).astype(jnp.float32)

    b0y1 = y1_ref[0, 0]
    b0x1 = x1_ref[0, 0]
    b0y2 = y2_ref[0, 0]
    b0x2 = x2_ref[0, 0]

    def body(i, _):
        s = sw_ref[...]
        m = jnp.max(s)
        idx = jnp.min(jnp.where(s == m, idxg, jnp.float32(_NP)))
        valid = (m > _NEG / 2.0) & (i < mos)
        sel = idxg == idx
        by1 = jnp.max(jnp.where(sel, y1_ref[...], _NEG))
        bx1 = jnp.max(jnp.where(sel, x1_ref[...], _NEG))
        by2 = jnp.max(jnp.where(sel, y2_ref[...], _NEG))
        bx2 = jnp.max(jnp.where(sel, x2_ref[...], _NEG))
        barea = jnp.max(jnp.where(sel, ar_ref[...], _NEG))

        yy1 = jnp.maximum(by1, y1_ref[...])
        xx1 = jnp.maximum(bx1, x1_ref[...])
        yy2 = jnp.minimum(by2, y2_ref[...])
        xx2 = jnp.minimum(bx2, x2_ref[...])
        inter = jnp.maximum(yy2 - yy1, 0.0) * jnp.maximum(xx2 - xx1, 0.0)
        union = barea + ar_ref[...] - inter
        iou = jnp.where(union > 0.0, inter / union, 0.0)
        supp = (iou >= 0.7) | sel
        sw_ref[...] = jnp.where(supp, _NEG, s)

        oy1 = jnp.where(valid, by1, b0y1)
        ox1 = jnp.where(valid, bx1, b0x1)
        oy2 = jnp.where(valid, by2, b0y2)
        ox2 = jnp.where(valid, bx2, b0x2)
        out_ref[pl.ds(i, 1), pl.ds(0, 1)] = jnp.full((1, 1), oy1)
        out_ref[pl.ds(i, 1), pl.ds(1, 1)] = jnp.full((1, 1), ox1)
        out_ref[pl.ds(i, 1), pl.ds(2, 1)] = jnp.full((1, 1), oy2)
        out_ref[pl.ds(i, 1), pl.ds(3, 1)] = jnp.full((1, 1), ox2)
        return 0

    lax.fori_loop(0, _MAXO, body, 0)


def kernel(rois, scores, max_output_size):
    s = jnp.squeeze(scores, axis=-1)
    s_p = jnp.concatenate([s, jnp.full((_NP - _N,), _NEG, jnp.float32)])
    zpad = jnp.zeros((_NP - _N,), jnp.float32)
    y1 = jnp.concatenate([rois[:, 0], zpad])
    x1 = jnp.concatenate([rois[:, 1], zpad])
    y2 = jnp.concatenate([rois[:, 2], zpad])
    x2 = jnp.concatenate([rois[:, 3], zpad])
    mos = jnp.asarray(max_output_size, jnp.int32).reshape(1)
    b0 = rois[0]

    cs, ci, cy1, cx1, cy2, cx2, thr = _sc_select(s_p, y1, x1, y2, x2)

    vspec = pl.BlockSpec(memory_space=pltpu.VMEM)
    sspec = pl.BlockSpec(memory_space=pltpu.SMEM)
    fast_out, flag = pl.pallas_call(
        _nms_cand_body,
        out_shape=[jax.ShapeDtypeStruct((_MAXO + 8, 4), jnp.float32),
                   jax.ShapeDtypeStruct((1,), jnp.int32)],
        in_specs=[sspec, sspec] + [vspec] * 7,
        out_specs=[vspec, sspec],
        scratch_shapes=[pltpu.VMEM((_CR, _CC), jnp.float32),
                        pltpu.VMEM((_CR, _CC), jnp.float32),
                        pltpu.SMEM((1,), jnp.int32)],
    )(mos, b0, thr.reshape(8, -1), cs.reshape(_CR, _CC),
      ci.astype(jnp.float32).reshape(_CR, _CC), cy1.reshape(_CR, _CC),
      cx1.reshape(_CR, _CC), cy2.reshape(_CR, _CC), cx2.reshape(_CR, _CC))

    def _full(_):
        return pl.pallas_call(
            _nms_full_body,
            out_shape=jax.ShapeDtypeStruct((_MAXO, 4), jnp.float32),
            in_specs=[sspec] + [vspec] * 5,
            scratch_shapes=[pltpu.VMEM((_R, _C), jnp.float32),
                            pltpu.VMEM((_R, _C), jnp.float32)],
        )(mos, s_p.reshape(_R, _C), y1.reshape(_R, _C), x1.reshape(_R, _C),
          y2.reshape(_R, _C), x2.reshape(_R, _C))

    def _fast(_):
        return fast_out[:_MAXO]

    return lax.cond(flag[0] > 0, _full, _fast, None)


# speedup vs baseline: 72.9530x; 1.0126x over previous
import functools

import numpy as np

import jax
import jax.numpy as jnp
from jax import lax
from jax.experimental import pallas as pl
from jax.experimental.pallas import tpu as pltpu
from jax.experimental.pallas import tpu_sc as plsc

_N = 20000
_NP = 20480
_NW = 32
_PT = _NP // _NW
_NG = _PT // 80
_M = _NG * 16
_NC = _NW * _M
_CR, _CC = 8, _NC // 8
_R, _C = 8, 2560
_MAXO = 1000
_NEG = float(np.float32(-1e30))


def _lexgt(s, i, sp, ip):
    return (s > sp) | ((s == sp) & (i < ip))


def _ce_stage(lane, s, idx, j, want_max):
    perm = lane ^ j
    sp = s[perm]
    ip = idx[perm]
    gt = _lexgt(s, idx, sp, ip)
    gi = jnp.where(gt, 1, 0)
    wi = jnp.where(want_max, 1, 0)
    take_self = (gi ^ wi) == 0
    return jnp.where(take_self, s, sp), jnp.where(take_self, idx, ip)


def _sort16_desc(lane, s, idx):
    for k in (2, 4, 8, 16):
        j = k // 2
        while j >= 1:
            jb = (lane >> j.bit_length() - 1) & 1
            kb = (lane >> k.bit_length() - 1) & 1
            want_max = jb == kb
            s, idx = _ce_stage(lane, s, idx, j, want_max)
            j //= 2
    return s, idx


def _merge_top16(lane, sa, ia, sb, ib):
    rperm = lane ^ 15
    rb = sb[rperm]
    rib = ib[rperm]
    gt = _lexgt(sa, ia, rb, rib)
    ts = jnp.where(gt, sa, rb)
    ti = jnp.where(gt, ia, rib)
    bs = jnp.where(gt, rb, sa)
    j = 8
    while j >= 1:
        want_max = (lane & j) == 0
        ts, ti = _ce_stage(lane, ts, ti, j, want_max)
        j //= 2
    bmax = bs
    for d in (8, 4, 2, 1):
        bmax = jnp.maximum(bmax, bmax[lane ^ d])
    return ts, ti, bmax


def _build_sc_select():
  mesh = plsc.VectorSubcoreMesh(core_axis_name="c", subcore_axis_name="s")

  @functools.partial(
    pl.kernel,
    mesh=mesh,
    out_type=[
        jax.ShapeDtypeStruct((_NC,), jnp.float32),
        jax.ShapeDtypeStruct((_NC,), jnp.int32),
        jax.ShapeDtypeStruct((_NC,), jnp.float32),
        jax.ShapeDtypeStruct((_NC,), jnp.float32),
        jax.ShapeDtypeStruct((_NC,), jnp.float32),
        jax.ShapeDtypeStruct((_NC,), jnp.float32),
        jax.ShapeDtypeStruct((_NW * 16,), jnp.float32),
    ],
    scratch_types=[
        pltpu.VMEM((_PT,), jnp.float32),
        pltpu.VMEM((_PT,), jnp.float32),
        pltpu.VMEM((_PT,), jnp.float32),
        pltpu.VMEM((_PT,), jnp.float32),
        pltpu.VMEM((_PT,), jnp.float32),
        pltpu.VMEM((_M,), jnp.float32),
        pltpu.VMEM((_M,), jnp.int32),
        pltpu.VMEM((_M,), jnp.float32),
        pltpu.VMEM((_M,), jnp.float32),
        pltpu.VMEM((_M,), jnp.float32),
        pltpu.VMEM((_M,), jnp.float32),
        pltpu.VMEM((16,), jnp.float32),
    ],
  )
  def _sc_sel(s_hbm, y1_hbm, x1_hbm, y2_hbm, x2_hbm,
              cs_out, ci_out, cy1_out, cx1_out, cy2_out, cx2_out, thr_out,
              s_v, y1_v, x1_v, y2_v, x2_v,
              cs_v, ci_v, cy1_v, cx1_v, cy2_v, cx2_v, thr_v):
    wid = lax.axis_index("s") * 2 + lax.axis_index("c")
    base = wid * _PT
    pltpu.sync_copy(s_hbm.at[pl.ds(base, _PT)], s_v)
    pltpu.sync_copy(y1_hbm.at[pl.ds(base, _PT)], y1_v)
    pltpu.sync_copy(x1_hbm.at[pl.ds(base, _PT)], x1_v)
    pltpu.sync_copy(y2_hbm.at[pl.ds(base, _PT)], y2_v)
    pltpu.sync_copy(x2_hbm.at[pl.ds(base, _PT)], x2_v)

    lane = lax.iota(jnp.int32, 16)
    neg = jnp.float32(_NEG)

    def _group(g, thrmax):
        goff = g * 80
        vs, vi = [], []
        for k in range(5):
            sv = s_v[pl.ds(goff + k * 16, 16)]
            gi = base + goff + k * 16 + lane
            ss, si = _sort16_desc(lane, sv, gi)
            vs.append(ss)
            vi.append(si)
        t01s, t01i, b01 = _merge_top16(lane, vs[0], vi[0], vs[1], vi[1])
        t23s, t23i, b23 = _merge_top16(lane, vs[2], vi[2], vs[3], vi[3])
        tqs, tqi, bq = _merge_top16(lane, t01s, t01i, t23s, t23i)
        ts, ti, bt = _merge_top16(lane, tqs, tqi, vs[4], vi[4])
        thr_g = jnp.maximum(jnp.maximum(b01, b23), jnp.maximum(bq, bt))
        thrmax = jnp.maximum(thrmax, thr_g)

        li = ti - (base + goff)
        vno = li >> 4
        lno = li - (vno << 4)
        outs = []
        for c_v in (y1_v, x1_v, y2_v, x2_v):
            r = jnp.zeros((16,), jnp.float32)
            for k in range(5):
                ck = c_v[pl.ds(goff + k * 16, 16)]
                r = jnp.where(vno == k, ck[lno], r)
            outs.append(r)

        cs_v[pl.ds(g * 16, 16)] = ts
        ci_v[pl.ds(g * 16, 16)] = ti
        cy1_v[pl.ds(g * 16, 16)] = outs[0]
        cx1_v[pl.ds(g * 16, 16)] = outs[1]
        cy2_v[pl.ds(g * 16, 16)] = outs[2]
        cx2_v[pl.ds(g * 16, 16)] = outs[3]
        return thrmax

    thrmax = lax.fori_loop(0, _NG, _group, jnp.full((16,), neg))
    thr_v[...] = thrmax

    pltpu.sync_copy(cs_v, cs_out.at[pl.ds(wid * _M, _M)])
    pltpu.sync_copy(ci_v, ci_out.at[pl.ds(wid * _M, _M)])
    pltpu.sync_copy(cy1_v, cy1_out.at[pl.ds(wid * _M, _M)])
    pltpu.sync_copy(cx1_v, cx1_out.at[pl.ds(wid * _M, _M)])
    pltpu.sync_copy(cy2_v, cy2_out.at[pl.ds(wid * _M, _M)])
    pltpu.sync_copy(cx2_v, cx2_out.at[pl.ds(wid * _M, _M)])
    pltpu.sync_copy(thr_v, thr_out.at[pl.ds(wid * 16, 16)])

  return _sc_sel


_sc_select_cached = None


def _sc_select(*args):
    global _sc_select_cached
    if _sc_select_cached is None:
        _sc_select_cached = _build_sc_select()
    return _sc_select_cached(*args)


def _scal_iou_ge(by1a, bx1a, by2a, bx2a, bara, by1b, bx1b, by2b, bx2b, barb):
    yy1 = jnp.maximum(by1a, by1b)
    xx1 = jnp.maximum(bx1a, bx1b)
    yy2 = jnp.minimum(by2a, by2b)
    xx2 = jnp.minimum(bx2a, bx2b)
    inter = jnp.maximum(yy2 - yy1, 0.0) * jnp.maximum(xx2 - xx1, 0.0)
    union = bara + barb - inter
    iou = jnp.where(union > 0.0, inter / union, 0.0)
    return iou >= 0.7


def _nms_cand_body(mos_ref, b0_ref, thr_ref, s_ref, idxf_ref,
                   y1_ref, x1_ref, y2_ref, x2_ref,
                   out_ref, flag_ref, sw_ref, ar_ref, acc_ref):
    sw_ref[...] = s_ref[...]
    ar_ref[...] = (jnp.maximum(y2_ref[...] - y1_ref[...], 0.0)
                   * jnp.maximum(x2_ref[...] - x1_ref[...], 0.0))
    mos = mos_ref[0]
    flag_ref[0] = (mos != _MAXO).astype(jnp.int32)
    mthr = jnp.max(thr_ref[...])
    b0y1 = b0_ref[0]
    b0x1 = b0_ref[1]
    b0y2 = b0_ref[2]
    b0x2 = b0_ref[3]
    out_ref[:, pl.ds(0, 1)] = jnp.full((_MAXO + 8, 1), b0y1)
    out_ref[:, pl.ds(1, 1)] = jnp.full((_MAXO + 8, 1), b0x1)
    out_ref[:, pl.ds(2, 1)] = jnp.full((_MAXO + 8, 1), b0y2)
    out_ref[:, pl.ds(3, 1)] = jnp.full((_MAXO + 8, 1), b0x2)
    row8 = lax.broadcasted_iota(jnp.int32, (8, 4), 0)
    col8 = lax.broadcasted_iota(jnp.int32, (8, 4), 1)

    neg = jnp.float32(_NEG)
    big = jnp.float32(_NP)

    def body(i, rows):
        acc_ref[0] = 0

        @pl.when(rows < mos)
        def _():
            s = sw_ref[...]
            idxp = idxf_ref[...]
            y1 = y1_ref[...]
            x1 = x1_ref[...]
            y2 = y2_ref[...]
            x2 = x2_ref[...]
            ar = ar_ref[...]

            m1 = jnp.max(s)
            q1 = s == m1
            s2 = jnp.where(q1, neg, s)
            m2 = jnp.max(s2)
            q2 = s2 == m2
            s3 = jnp.where(q2, neg, s2)
            m3 = jnp.max(s3)
            q3 = s3 == m3
            s4 = jnp.where(q3, neg, s3)
            m4 = jnp.max(s4)
            q4 = s4 == m4

            idx1 = jnp.min(jnp.where(q1, idxp, big))
            sel1 = idxp == idx1
            ones = jnp.float32(1.0)
            cnt1 = jnp.sum(jnp.where(q1, ones, 0.0))
            cnt2 = jnp.sum(jnp.where(q2, ones, 0.0))
            cnt3 = jnp.sum(jnp.where(q3, ones, 0.0))
            cnt4 = jnp.sum(jnp.where(q4, ones, 0.0))

            def extract(sel):
                return (jnp.max(jnp.where(sel, y1, neg)),
                        jnp.max(jnp.where(sel, x1, neg)),
                        jnp.max(jnp.where(sel, y2, neg)),
                        jnp.max(jnp.where(sel, x2, neg)),
                        jnp.max(jnp.where(sel, ar, neg)))

            c1 = extract(sel1)
            c2 = extract(q2)
            c3 = extract(q3)
            c4 = extract(q4)

            ok2 = (cnt1 <= 1.0) & (cnt2 <= 1.0) & (m2 > _NEG / 2.0)
            ok3 = ok2 & (cnt3 <= 1.0) & (m3 > _NEG / 2.0)
            ok4 = ok3 & (cnt4 <= 1.0) & (m4 > _NEG / 2.0)


            a2 = ok2 & ~_scal_iou_ge(*c1, *c2)
            a3 = ok3 & ~((a2 & _scal_iou_ge(*c2, *c3))
                         | _scal_iou_ge(*c1, *c3))
            a4 = ok4 & ~((a3 & _scal_iou_ge(*c3, *c4))
                         | (a2 & _scal_iou_ge(*c2, *c4))
                         | _scal_iou_ge(*c1, *c4))

            r1 = rows
            r2 = r1 + 1
            r3 = r2 + jnp.where(a2, 1, 0)
            r4 = r3 + jnp.where(a3, 1, 0)
            w1 = r1 < mos
            w2 = a2 & (r2 < mos)
            w3 = a3 & (r3 < mos)
            w4 = a4 & (r4 < mos)

            mlast = jnp.where(w4, m4, jnp.where(w3, m3,
                              jnp.where(w2, m2, m1)))
            flag_ref[0] = flag_ref[0] | (mlast <= mthr).astype(jnp.int32) \
                | (m1 <= mthr).astype(jnp.int32)

            blk = jnp.where(col8 == 0, b0y1,
                  jnp.where(col8 == 1, b0x1,
                  jnp.where(col8 == 2, b0y2, b0x2)))
            for w, r, c in ((w1, r1, c1), (w2, r2, c2),
                            (w3, r3, c3), (w4, r4, c4)):
                hit = w & (row8 == (r - rows))
                blk = jnp.where(hit & (col8 == 0), c[0],
                      jnp.where(hit & (col8 == 1), c[1],
                      jnp.where(hit & (col8 == 2), c[2],
                      jnp.where(hit & (col8 == 3), c[3], blk))))
            out_ref[pl.ds(rows, 8), :] = blk

            def supp_mask(c, selq):
                yy1 = jnp.maximum(c[0], y1)
                xx1 = jnp.maximum(c[1], x1)
                yy2 = jnp.minimum(c[2], y2)
                xx2 = jnp.minimum(c[3], x2)
                inter = (jnp.maximum(yy2 - yy1, 0.0)
                         * jnp.maximum(xx2 - xx1, 0.0))
                union = c[4] + ar - inter
                iou = jnp.where(union > 0.0, inter / union, 0.0)
                return (iou >= 0.7) | selq

            sm = supp_mask(c1, sel1)
            sm = sm | (w2 & supp_mask(c2, q2))
            sm = sm | (w3 & supp_mask(c3, q3))
            sm = sm | (w4 & supp_mask(c4, q4))
            sw_ref[...] = jnp.where(sm, neg, s)

            acc_ref[0] = (jnp.where(w1, 1, 0) + jnp.where(w2, 1, 0)
                          + jnp.where(w3, 1, 0) + jnp.where(w4, 1, 0))

        return rows + acc_ref[0]

    rows = lax.fori_loop(0, 320, body, jnp.int32(0))
    flag_ref[0] = flag_ref[0] | (rows < mos).astype(jnp.int32)


def _nms_full_body(mos_ref, s_ref, y1_ref, x1_ref, y2_ref, x2_ref,
                   out_ref, sw_ref, ar_ref):
    sw_ref[...] = s_ref[...]
    ar_ref[...] = (jnp.maximum(y2_ref[...] - y1_ref[...], 0.0)
                   * jnp.maximum(x2_ref[...] - x1_ref[...], 0.0))
    mos = mos_ref[0]
    rows = lax.broadcasted_iota(jnp.int32, (_R, _C), 0)
    cols = lax.broadcasted_iota(jnp.int32, (_R, _C), 1)
    idxg = (rows * _C + cols).astype(jnp.float32)

    b0y1 = y1_ref[0, 0]
    b0x1 = x1_ref[0, 0]
    b0y2 = y2_ref[0, 0]
    b0x2 = x2_ref[0, 0]

    def body(i, _):
        s = sw_ref[...]
        m = jnp.max(s)
        idx = jnp.min(jnp.where(s == m, idxg, jnp.float32(_NP)))
        valid = (m > _NEG / 2.0) & (i < mos)
        sel = idxg == idx
        by1 = jnp.max(jnp.where(sel, y1_ref[...], _NEG))
        bx1 = jnp.max(jnp.where(sel, x1_ref[...], _NEG))
        by2 = jnp.max(jnp.where(sel, y2_ref[...], _NEG))
        bx2 = jnp.max(jnp.where(sel, x2_ref[...], _NEG))
        barea = jnp.max(jnp.where(sel, ar_ref[...], _NEG))

        yy1 = jnp.maximum(by1, y1_ref[...])
        xx1 = jnp.maximum(bx1, x1_ref[...])
        yy2 = jnp.minimum(by2, y2_ref[...])
        xx2 = jnp.minimum(bx2, x2_ref[...])
        inter = jnp.maximum(yy2 - yy1, 0.0) * jnp.maximum(xx2 - xx1, 0.0)
        union = barea + ar_ref[...] - inter
        iou = jnp.where(union > 0.0, inter / union, 0.0)
        supp = (iou >= 0.7) | sel
        sw_ref[...] = jnp.where(supp, _NEG, s)

        oy1 = jnp.where(valid, by1, b0y1)
        ox1 = jnp.where(valid, bx1, b0x1)
        oy2 = jnp.where(valid, by2, b0y2)
        ox2 = jnp.where(valid, bx2, b0x2)
        out_ref[pl.ds(i, 1), pl.ds(0, 1)] = jnp.full((1, 1), oy1)
        out_ref[pl.ds(i, 1), pl.ds(1, 1)] = jnp.full((1, 1), ox1)
        out_ref[pl.ds(i, 1), pl.ds(2, 1)] = jnp.full((1, 1), oy2)
        out_ref[pl.ds(i, 1), pl.ds(3, 1)] = jnp.full((1, 1), ox2)
        return 0

    lax.fori_loop(0, _MAXO, body, 0)


def kernel(rois, scores, max_output_size):
    s = jnp.squeeze(scores, axis=-1)
    s_p = jnp.concatenate([s, jnp.full((_NP - _N,), _NEG, jnp.float32)])
    zpad = jnp.zeros((_NP - _N,), jnp.float32)
    y1 = jnp.concatenate([rois[:, 0], zpad])
    x1 = jnp.concatenate([rois[:, 1], zpad])
    y2 = jnp.concatenate([rois[:, 2], zpad])
    x2 = jnp.concatenate([rois[:, 3], zpad])
    mos = jnp.asarray(max_output_size, jnp.int32).reshape(1)
    b0 = rois[0]

    cs, ci, cy1, cx1, cy2, cx2, thr = _sc_select(s_p, y1, x1, y2, x2)

    vspec = pl.BlockSpec(memory_space=pltpu.VMEM)
    sspec = pl.BlockSpec(memory_space=pltpu.SMEM)
    fast_out, flag = pl.pallas_call(
        _nms_cand_body,
        out_shape=[jax.ShapeDtypeStruct((_MAXO + 8, 4), jnp.float32),
                   jax.ShapeDtypeStruct((1,), jnp.int32)],
        in_specs=[sspec, sspec] + [vspec] * 7,
        out_specs=[vspec, sspec],
        scratch_shapes=[pltpu.VMEM((_CR, _CC), jnp.float32),
                        pltpu.VMEM((_CR, _CC), jnp.float32),
                        pltpu.SMEM((1,), jnp.int32)],
    )(mos, b0, thr.reshape(8, -1), cs.reshape(_CR, _CC),
      ci.astype(jnp.float32).reshape(_CR, _CC), cy1.reshape(_CR, _CC),
      cx1.reshape(_CR, _CC), cy2.reshape(_CR, _CC), cx2.reshape(_CR, _CC))

    def _full(_):
        return pl.pallas_call(
            _nms_full_body,
            out_shape=jax.ShapeDtypeStruct((_MAXO, 4), jnp.float32),
            in_specs=[sspec] + [vspec] * 5,
            scratch_shapes=[pltpu.VMEM((_R, _C), jnp.float32),
                            pltpu.VMEM((_R, _C), jnp.float32)],
        )(mos, s_p.reshape(_R, _C), y1.reshape(_R, _C), x1.reshape(_R, _C),
          y2.reshape(_R, _C), x2.reshape(_R, _C))

    def _fast(_):
        return fast_out[:_MAXO]

    return lax.cond(flag[0] > 0, _full, _fast, None)


# speedup vs baseline: 75.2669x; 1.0317x over previous
import functools

import numpy as np

import jax
import jax.numpy as jnp
from jax import lax
from jax.experimental import pallas as pl
from jax.experimental.pallas import tpu as pltpu
from jax.experimental.pallas import tpu_sc as plsc

_N = 20000
_NP = 20480
_NW = 32
_PT = _NP // _NW
_NG = _PT // 80
_M = _NG * 16
_NC = _NW * _M
_CR, _CC = 8, _NC // 8
_R, _C = 8, 2560
_MAXO = 1000
_NEG = float(np.float32(-1e30))


def _lexgt(s, i, sp, ip):
    return (s > sp) | ((s == sp) & (i < ip))


def _ce_stage(lane, s, idx, j, want_max):
    perm = lane ^ j
    sp = s[perm]
    ip = idx[perm]
    gt = _lexgt(s, idx, sp, ip)
    gi = jnp.where(gt, 1, 0)
    wi = jnp.where(want_max, 1, 0)
    take_self = (gi ^ wi) == 0
    return jnp.where(take_self, s, sp), jnp.where(take_self, idx, ip)


def _sort16_desc(lane, s, idx):
    for k in (2, 4, 8, 16):
        j = k // 2
        while j >= 1:
            jb = (lane >> j.bit_length() - 1) & 1
            kb = (lane >> k.bit_length() - 1) & 1
            want_max = jb == kb
            s, idx = _ce_stage(lane, s, idx, j, want_max)
            j //= 2
    return s, idx


def _merge_top16(lane, sa, ia, sb, ib):
    rperm = lane ^ 15
    rb = sb[rperm]
    rib = ib[rperm]
    gt = _lexgt(sa, ia, rb, rib)
    ts = jnp.where(gt, sa, rb)
    ti = jnp.where(gt, ia, rib)
    bs = jnp.where(gt, rb, sa)
    j = 8
    while j >= 1:
        want_max = (lane & j) == 0
        ts, ti = _ce_stage(lane, ts, ti, j, want_max)
        j //= 2
    bmax = bs
    for d in (8, 4, 2, 1):
        bmax = jnp.maximum(bmax, bmax[lane ^ d])
    return ts, ti, bmax


def _build_sc_select():
  mesh = plsc.VectorSubcoreMesh(core_axis_name="c", subcore_axis_name="s")

  @functools.partial(
    pl.kernel,
    mesh=mesh,
    out_type=[
        jax.ShapeDtypeStruct((_NC,), jnp.float32),
        jax.ShapeDtypeStruct((_NC,), jnp.int32),
        jax.ShapeDtypeStruct((_NC,), jnp.float32),
        jax.ShapeDtypeStruct((_NC,), jnp.float32),
        jax.ShapeDtypeStruct((_NC,), jnp.float32),
        jax.ShapeDtypeStruct((_NC,), jnp.float32),
        jax.ShapeDtypeStruct((_NW * 16,), jnp.float32),
    ],
    scratch_types=[
        pltpu.VMEM((_PT,), jnp.float32),
        pltpu.VMEM((_PT,), jnp.float32),
        pltpu.VMEM((_PT,), jnp.float32),
        pltpu.VMEM((_PT,), jnp.float32),
        pltpu.VMEM((_PT,), jnp.float32),
        pltpu.VMEM((_M,), jnp.float32),
        pltpu.VMEM((_M,), jnp.int32),
        pltpu.VMEM((_M,), jnp.float32),
        pltpu.VMEM((_M,), jnp.float32),
        pltpu.VMEM((_M,), jnp.float32),
        pltpu.VMEM((_M,), jnp.float32),
        pltpu.VMEM((16,), jnp.float32),
    ],
  )
  def _sc_sel(s_hbm, y1_hbm, x1_hbm, y2_hbm, x2_hbm,
              cs_out, ci_out, cy1_out, cx1_out, cy2_out, cx2_out, thr_out,
              s_v, y1_v, x1_v, y2_v, x2_v,
              cs_v, ci_v, cy1_v, cx1_v, cy2_v, cx2_v, thr_v):
    wid = lax.axis_index("s") * 2 + lax.axis_index("c")
    base = wid * _PT
    pltpu.sync_copy(s_hbm.at[pl.ds(base, _PT)], s_v)
    pltpu.sync_copy(y1_hbm.at[pl.ds(base, _PT)], y1_v)
    pltpu.sync_copy(x1_hbm.at[pl.ds(base, _PT)], x1_v)
    pltpu.sync_copy(y2_hbm.at[pl.ds(base, _PT)], y2_v)
    pltpu.sync_copy(x2_hbm.at[pl.ds(base, _PT)], x2_v)

    lane = lax.iota(jnp.int32, 16)
    neg = jnp.float32(_NEG)

    def _group(g, thrmax):
        goff = g * 80
        vs, vi = [], []
        for k in range(5):
            sv = s_v[pl.ds(goff + k * 16, 16)]
            gi = base + goff + k * 16 + lane
            ss, si = _sort16_desc(lane, sv, gi)
            vs.append(ss)
            vi.append(si)
        t01s, t01i, b01 = _merge_top16(lane, vs[0], vi[0], vs[1], vi[1])
        t23s, t23i, b23 = _merge_top16(lane, vs[2], vi[2], vs[3], vi[3])
        tqs, tqi, bq = _merge_top16(lane, t01s, t01i, t23s, t23i)
        ts, ti, bt = _merge_top16(lane, tqs, tqi, vs[4], vi[4])
        thr_g = jnp.maximum(jnp.maximum(b01, b23), jnp.maximum(bq, bt))
        thrmax = jnp.maximum(thrmax, thr_g)

        li = ti - (base + goff)
        vno = li >> 4
        lno = li - (vno << 4)
        outs = []
        for c_v in (y1_v, x1_v, y2_v, x2_v):
            r = jnp.zeros((16,), jnp.float32)
            for k in range(5):
                ck = c_v[pl.ds(goff + k * 16, 16)]
                r = jnp.where(vno == k, ck[lno], r)
            outs.append(r)

        cs_v[pl.ds(g * 16, 16)] = ts
        ci_v[pl.ds(g * 16, 16)] = ti
        cy1_v[pl.ds(g * 16, 16)] = outs[0]
        cx1_v[pl.ds(g * 16, 16)] = outs[1]
        cy2_v[pl.ds(g * 16, 16)] = outs[2]
        cx2_v[pl.ds(g * 16, 16)] = outs[3]
        return thrmax

    thrmax = lax.fori_loop(0, _NG, _group, jnp.full((16,), neg))
    thr_v[...] = thrmax

    pltpu.sync_copy(cs_v, cs_out.at[pl.ds(wid * _M, _M)])
    pltpu.sync_copy(ci_v, ci_out.at[pl.ds(wid * _M, _M)])
    pltpu.sync_copy(cy1_v, cy1_out.at[pl.ds(wid * _M, _M)])
    pltpu.sync_copy(cx1_v, cx1_out.at[pl.ds(wid * _M, _M)])
    pltpu.sync_copy(cy2_v, cy2_out.at[pl.ds(wid * _M, _M)])
    pltpu.sync_copy(cx2_v, cx2_out.at[pl.ds(wid * _M, _M)])
    pltpu.sync_copy(thr_v, thr_out.at[pl.ds(wid * 16, 16)])

  return _sc_sel


_sc_select_cached = None


def _sc_select(*args):
    global _sc_select_cached
    if _sc_select_cached is None:
        _sc_select_cached = _build_sc_select()
    return _sc_select_cached(*args)


def _scal_iou_ge(by1a, bx1a, by2a, bx2a, bara, by1b, bx1b, by2b, bx2b, barb):
    yy1 = jnp.maximum(by1a, by1b)
    xx1 = jnp.maximum(bx1a, bx1b)
    yy2 = jnp.minimum(by2a, by2b)
    xx2 = jnp.minimum(bx2a, bx2b)
    inter = jnp.maximum(yy2 - yy1, 0.0) * jnp.maximum(xx2 - xx1, 0.0)
    union = bara + barb - inter
    iou = jnp.where(union > 0.0, inter / union, 0.0)
    return iou >= 0.7


def _nms_cand_body(mos_ref, b0_ref, thr_ref, s_ref, idxf_ref,
                   y1_ref, x1_ref, y2_ref, x2_ref,
                   out_ref, flag_ref, sw_ref, ar_ref, acc_ref):
    sw_ref[...] = s_ref[...]
    ar_ref[...] = (jnp.maximum(y2_ref[...] - y1_ref[...], 0.0)
                   * jnp.maximum(x2_ref[...] - x1_ref[...], 0.0))
    mos = mos_ref[0]
    flag_ref[0] = (mos != _MAXO).astype(jnp.int32)
    mthr = jnp.max(thr_ref[...])
    b0y1 = b0_ref[0]
    b0x1 = b0_ref[1]
    b0y2 = b0_ref[2]
    b0x2 = b0_ref[3]
    out_ref[:, pl.ds(0, 1)] = jnp.full((_MAXO + 8, 1), b0y1)
    out_ref[:, pl.ds(1, 1)] = jnp.full((_MAXO + 8, 1), b0x1)
    out_ref[:, pl.ds(2, 1)] = jnp.full((_MAXO + 8, 1), b0y2)
    out_ref[:, pl.ds(3, 1)] = jnp.full((_MAXO + 8, 1), b0x2)
    row8 = lax.broadcasted_iota(jnp.int32, (8, 4), 0)
    col8 = lax.broadcasted_iota(jnp.int32, (8, 4), 1)

    neg = jnp.float32(_NEG)
    big = jnp.float32(_NP)

    def body(i, rows):
        acc_ref[0] = 0

        @pl.when(rows < mos)
        def _():
            s = sw_ref[...]
            idxp = idxf_ref[...]
            y1 = y1_ref[...]
            x1 = x1_ref[...]
            y2 = y2_ref[...]
            x2 = x2_ref[...]
            ar = ar_ref[...]

            m1 = jnp.max(s)
            q1 = s == m1
            s2 = jnp.where(q1, neg, s)
            m2 = jnp.max(s2)
            q2 = s2 == m2
            s3 = jnp.where(q2, neg, s2)
            m3 = jnp.max(s3)
            q3 = s3 == m3
            s4 = jnp.where(q3, neg, s3)
            m4 = jnp.max(s4)
            q4 = s4 == m4
            s5 = jnp.where(q4, neg, s4)
            m5 = jnp.max(s5)
            q5 = s5 == m5
            s6 = jnp.where(q5, neg, s5)
            m6 = jnp.max(s6)
            q6 = s6 == m6

            idx1 = jnp.min(jnp.where(q1, idxp, big))
            sel1 = idxp == idx1
            ones = jnp.float32(1.0)
            cnt1 = jnp.sum(jnp.where(q1, ones, 0.0))
            cnt2 = jnp.sum(jnp.where(q2, ones, 0.0))
            cnt3 = jnp.sum(jnp.where(q3, ones, 0.0))
            cnt4 = jnp.sum(jnp.where(q4, ones, 0.0))
            cnt5 = jnp.sum(jnp.where(q5, ones, 0.0))
            cnt6 = jnp.sum(jnp.where(q6, ones, 0.0))

            def extract(sel):
                return (jnp.max(jnp.where(sel, y1, neg)),
                        jnp.max(jnp.where(sel, x1, neg)),
                        jnp.max(jnp.where(sel, y2, neg)),
                        jnp.max(jnp.where(sel, x2, neg)),
                        jnp.max(jnp.where(sel, ar, neg)))

            c1 = extract(sel1)
            c2 = extract(q2)
            c3 = extract(q3)
            c4 = extract(q4)
            c5 = extract(q5)
            c6 = extract(q6)

            ok2 = (cnt1 <= 1.0) & (cnt2 <= 1.0) & (m2 > _NEG / 2.0)
            ok3 = ok2 & (cnt3 <= 1.0) & (m3 > _NEG / 2.0)
            ok4 = ok3 & (cnt4 <= 1.0) & (m4 > _NEG / 2.0)
            ok5 = ok4 & (cnt5 <= 1.0) & (m5 > _NEG / 2.0)
            ok6 = ok5 & (cnt6 <= 1.0) & (m6 > _NEG / 2.0)


            a2 = ok2 & ~_scal_iou_ge(*c1, *c2)
            a3 = ok3 & ~((a2 & _scal_iou_ge(*c2, *c3))
                         | _scal_iou_ge(*c1, *c3))
            a4 = ok4 & ~((a3 & _scal_iou_ge(*c3, *c4))
                         | (a2 & _scal_iou_ge(*c2, *c4))
                         | _scal_iou_ge(*c1, *c4))
            a5 = ok5 & ~((a4 & _scal_iou_ge(*c4, *c5))
                         | (a3 & _scal_iou_ge(*c3, *c5))
                         | (a2 & _scal_iou_ge(*c2, *c5))
                         | _scal_iou_ge(*c1, *c5))
            a6 = ok6 & ~((a5 & _scal_iou_ge(*c5, *c6))
                         | (a4 & _scal_iou_ge(*c4, *c6))
                         | (a3 & _scal_iou_ge(*c3, *c6))
                         | (a2 & _scal_iou_ge(*c2, *c6))
                         | _scal_iou_ge(*c1, *c6))

            r1 = rows
            r2 = r1 + 1
            r3 = r2 + jnp.where(a2, 1, 0)
            r4 = r3 + jnp.where(a3, 1, 0)
            r5 = r4 + jnp.where(a4, 1, 0)
            r6 = r5 + jnp.where(a5, 1, 0)
            w1 = r1 < mos
            w2 = a2 & (r2 < mos)
            w3 = a3 & (r3 < mos)
            w4 = a4 & (r4 < mos)
            w5 = a5 & (r5 < mos)
            w6 = a6 & (r6 < mos)

            mlast = jnp.where(w6, m6, jnp.where(w5, m5,
                    jnp.where(w4, m4, jnp.where(w3, m3,
                              jnp.where(w2, m2, m1)))))
            flag_ref[0] = flag_ref[0] | (mlast <= mthr).astype(jnp.int32) \
                | (m1 <= mthr).astype(jnp.int32)

            blk = jnp.where(col8 == 0, b0y1,
                  jnp.where(col8 == 1, b0x1,
                  jnp.where(col8 == 2, b0y2, b0x2)))
            for w, r, c in ((w1, r1, c1), (w2, r2, c2),
                            (w3, r3, c3), (w4, r4, c4),
                            (w5, r5, c5), (w6, r6, c6)):
                hit = w & (row8 == (r - rows))
                blk = jnp.where(hit & (col8 == 0), c[0],
                      jnp.where(hit & (col8 == 1), c[1],
                      jnp.where(hit & (col8 == 2), c[2],
                      jnp.where(hit & (col8 == 3), c[3], blk))))
            out_ref[pl.ds(rows, 8), :] = blk

            def supp_mask(c, selq):
                yy1 = jnp.maximum(c[0], y1)
                xx1 = jnp.maximum(c[1], x1)
                yy2 = jnp.minimum(c[2], y2)
                xx2 = jnp.minimum(c[3], x2)
                inter = (jnp.maximum(yy2 - yy1, 0.0)
                         * jnp.maximum(xx2 - xx1, 0.0))
                union = c[4] + ar - inter
                iou = jnp.where(union > 0.0, inter / union, 0.0)
                return (iou >= 0.7) | selq

            sm = supp_mask(c1, sel1)
            sm = sm | (w2 & supp_mask(c2, q2))
            sm = sm | (w3 & supp_mask(c3, q3))
            sm = sm | (w4 & supp_mask(c4, q4))
            sm = sm | (w5 & supp_mask(c5, q5))
            sm = sm | (w6 & supp_mask(c6, q6))
            sw_ref[...] = jnp.where(sm, neg, s)

            acc_ref[0] = (jnp.where(w1, 1, 0) + jnp.where(w2, 1, 0)
                          + jnp.where(w3, 1, 0) + jnp.where(w4, 1, 0)
                          + jnp.where(w5, 1, 0) + jnp.where(w6, 1, 0))

        return rows + acc_ref[0]

    rows = lax.fori_loop(0, 224, body, jnp.int32(0))
    flag_ref[0] = flag_ref[0] | (rows < mos).astype(jnp.int32)


def _nms_full_body(mos_ref, s_ref, y1_ref, x1_ref, y2_ref, x2_ref,
                   out_ref, sw_ref, ar_ref):
    sw_ref[...] = s_ref[...]
    ar_ref[...] = (jnp.maximum(y2_ref[...] - y1_ref[...], 0.0)
                   * jnp.maximum(x2_ref[...] - x1_ref[...], 0.0))
    mos = mos_ref[0]
    rows = lax.broadcasted_iota(jnp.int32, (_R, _C), 0)
    cols = lax.broadcasted_iota(jnp.int32, (_R, _C), 1)
    idxg = (rows * _C + cols).astype(jnp.float32)

    b0y1 = y1_ref[0, 0]
    b0x1 = x1_ref[0, 0]
    b0y2 = y2_ref[0, 0]
    b0x2 = x2_ref[0, 0]

    def body(i, _):
        s = sw_ref[...]
        m = jnp.max(s)
        idx = jnp.min(jnp.where(s == m, idxg, jnp.float32(_NP)))
        valid = (m > _NEG / 2.0) & (i < mos)
        sel = idxg == idx
        by1 = jnp.max(jnp.where(sel, y1_ref[...], _NEG))
        bx1 = jnp.max(jnp.where(sel, x1_ref[...], _NEG))
        by2 = jnp.max(jnp.where(sel, y2_ref[...], _NEG))
        bx2 = jnp.max(jnp.where(sel, x2_ref[...], _NEG))
        barea = jnp.max(jnp.where(sel, ar_ref[...], _NEG))

        yy1 = jnp.maximum(by1, y1_ref[...])
        xx1 = jnp.maximum(bx1, x1_ref[...])
        yy2 = jnp.minimum(by2, y2_ref[...])
        xx2 = jnp.minimum(bx2, x2_ref[...])
        inter = jnp.maximum(yy2 - yy1, 0.0) * jnp.maximum(xx2 - xx1, 0.0)
        union = barea + ar_ref[...] - inter
        iou = jnp.where(union > 0.0, inter / union, 0.0)
        supp = (iou >= 0.7) | sel
        sw_ref[...] = jnp.where(supp, _NEG, s)

        oy1 = jnp.where(valid, by1, b0y1)
        ox1 = jnp.where(valid, bx1, b0x1)
        oy2 = jnp.where(valid, by2, b0y2)
        ox2 = jnp.where(valid, bx2, b0x2)
        out_ref[pl.ds(i, 1), pl.ds(0, 1)] = jnp.full((1, 1), oy1)
        out_ref[pl.ds(i, 1), pl.ds(1, 1)] = jnp.full((1, 1), ox1)
        out_ref[pl.ds(i, 1), pl.ds(2, 1)] = jnp.full((1, 1), oy2)
        out_ref[pl.ds(i, 1), pl.ds(3, 1)] = jnp.full((1, 1), ox2)
        return 0

    lax.fori_loop(0, _MAXO, body, 0)


def kernel(rois, scores, max_output_size):
    s = jnp.squeeze(scores, axis=-1)
    s_p = jnp.concatenate([s, jnp.full((_NP - _N,), _NEG, jnp.float32)])
    zpad = jnp.zeros((_NP - _N,), jnp.float32)
    y1 = jnp.concatenate([rois[:, 0], zpad])
    x1 = jnp.concatenate([rois[:, 1], zpad])
    y2 = jnp.concatenate([rois[:, 2], zpad])
    x2 = jnp.concatenate([rois[:, 3], zpad])
    mos = jnp.asarray(max_output_size, jnp.int32).reshape(1)
    b0 = rois[0]

    cs, ci, cy1, cx1, cy2, cx2, thr = _sc_select(s_p, y1, x1, y2, x2)

    vspec = pl.BlockSpec(memory_space=pltpu.VMEM)
    sspec = pl.BlockSpec(memory_space=pltpu.SMEM)
    fast_out, flag = pl.pallas_call(
        _nms_cand_body,
        out_shape=[jax.ShapeDtypeStruct((_MAXO + 8, 4), jnp.float32),
                   jax.ShapeDtypeStruct((1,), jnp.int32)],
        in_specs=[sspec, sspec] + [vspec] * 7,
        out_specs=[vspec, sspec],
        scratch_shapes=[pltpu.VMEM((_CR, _CC), jnp.float32),
                        pltpu.VMEM((_CR, _CC), jnp.float32),
                        pltpu.SMEM((1,), jnp.int32)],
    )(mos, b0, thr.reshape(8, -1), cs.reshape(_CR, _CC),
      ci.astype(jnp.float32).reshape(_CR, _CC), cy1.reshape(_CR, _CC),
      cx1.reshape(_CR, _CC), cy2.reshape(_CR, _CC), cx2.reshape(_CR, _CC))

    def _full(_):
        return pl.pallas_call(
            _nms_full_body,
            out_shape=jax.ShapeDtypeStruct((_MAXO, 4), jnp.float32),
            in_specs=[sspec] + [vspec] * 5,
            scratch_shapes=[pltpu.VMEM((_R, _C), jnp.float32),
                            pltpu.VMEM((_R, _C), jnp.float32)],
        )(mos, s_p.reshape(_R, _C), y1.reshape(_R, _C), x1.reshape(_R, _C),
          y2.reshape(_R, _C), x2.reshape(_R, _C))

    def _fast(_):
        return fast_out[:_MAXO]

    return lax.cond(flag[0] > 0, _full, _fast, None)


# speedup vs baseline: 75.3295x; 1.0008x over previous
import functools

import numpy as np

import jax
import jax.numpy as jnp
from jax import lax
from jax.experimental import pallas as pl
from jax.experimental.pallas import tpu as pltpu
from jax.experimental.pallas import tpu_sc as plsc

_N = 20000
_NP = 20480
_NW = 32
_PT = _NP // _NW
_NG = _PT // 80
_M = _NG * 16
_NC = _NW * _M
_CR, _CC = 8, _NC // 8
_R, _C = 8, 2560
_MAXO = 1000
_NEG = float(np.float32(-1e30))


def _lexgt(s, i, sp, ip):
    return (s > sp) | ((s == sp) & (i < ip))


def _ce_stage(lane, s, idx, j, want_max):
    perm = lane ^ j
    sp = s[perm]
    ip = idx[perm]
    gt = _lexgt(s, idx, sp, ip)
    gi = jnp.where(gt, 1, 0)
    wi = jnp.where(want_max, 1, 0)
    take_self = (gi ^ wi) == 0
    return jnp.where(take_self, s, sp), jnp.where(take_self, idx, ip)


def _sort16_desc(lane, s, idx):
    for k in (2, 4, 8, 16):
        j = k // 2
        while j >= 1:
            jb = (lane >> j.bit_length() - 1) & 1
            kb = (lane >> k.bit_length() - 1) & 1
            want_max = jb == kb
            s, idx = _ce_stage(lane, s, idx, j, want_max)
            j //= 2
    return s, idx


def _merge_top16(lane, sa, ia, sb, ib):
    rperm = lane ^ 15
    rb = sb[rperm]
    rib = ib[rperm]
    gt = _lexgt(sa, ia, rb, rib)
    ts = jnp.where(gt, sa, rb)
    ti = jnp.where(gt, ia, rib)
    bs = jnp.where(gt, rb, sa)
    j = 8
    while j >= 1:
        want_max = (lane & j) == 0
        ts, ti = _ce_stage(lane, ts, ti, j, want_max)
        j //= 2
    bmax = bs
    for d in (8, 4, 2, 1):
        bmax = jnp.maximum(bmax, bmax[lane ^ d])
    return ts, ti, bmax


def _build_sc_select():
  mesh = plsc.VectorSubcoreMesh(core_axis_name="c", subcore_axis_name="s")

  @functools.partial(
    pl.kernel,
    mesh=mesh,
    out_type=[
        jax.ShapeDtypeStruct((_NC,), jnp.float32),
        jax.ShapeDtypeStruct((_NC,), jnp.int32),
        jax.ShapeDtypeStruct((_NC,), jnp.float32),
        jax.ShapeDtypeStruct((_NC,), jnp.float32),
        jax.ShapeDtypeStruct((_NC,), jnp.float32),
        jax.ShapeDtypeStruct((_NC,), jnp.float32),
        jax.ShapeDtypeStruct((_NW * 16,), jnp.float32),
    ],
    scratch_types=[
        pltpu.VMEM((_PT,), jnp.float32),
        pltpu.VMEM((_PT,), jnp.float32),
        pltpu.VMEM((_PT,), jnp.float32),
        pltpu.VMEM((_PT,), jnp.float32),
        pltpu.VMEM((_PT,), jnp.float32),
        pltpu.VMEM((_M,), jnp.float32),
        pltpu.VMEM((_M,), jnp.int32),
        pltpu.VMEM((_M,), jnp.float32),
        pltpu.VMEM((_M,), jnp.float32),
        pltpu.VMEM((_M,), jnp.float32),
        pltpu.VMEM((_M,), jnp.float32),
        pltpu.VMEM((16,), jnp.float32),
    ],
  )
  def _sc_sel(s_hbm, y1_hbm, x1_hbm, y2_hbm, x2_hbm,
              cs_out, ci_out, cy1_out, cx1_out, cy2_out, cx2_out, thr_out,
              s_v, y1_v, x1_v, y2_v, x2_v,
              cs_v, ci_v, cy1_v, cx1_v, cy2_v, cx2_v, thr_v):
    wid = lax.axis_index("s") * 2 + lax.axis_index("c")
    base = wid * _PT
    pltpu.sync_copy(s_hbm.at[pl.ds(base, _PT)], s_v)
    pltpu.sync_copy(y1_hbm.at[pl.ds(base, _PT)], y1_v)
    pltpu.sync_copy(x1_hbm.at[pl.ds(base, _PT)], x1_v)
    pltpu.sync_copy(y2_hbm.at[pl.ds(base, _PT)], y2_v)
    pltpu.sync_copy(x2_hbm.at[pl.ds(base, _PT)], x2_v)

    lane = lax.iota(jnp.int32, 16)
    neg = jnp.float32(_NEG)

    def _group(g, thrmax):
        goff = g * 80
        vs, vi = [], []
        for k in range(5):
            sv = s_v[pl.ds(goff + k * 16, 16)]
            gi = base + goff + k * 16 + lane
            ss, si = _sort16_desc(lane, sv, gi)
            vs.append(ss)
            vi.append(si)
        t01s, t01i, b01 = _merge_top16(lane, vs[0], vi[0], vs[1], vi[1])
        t23s, t23i, b23 = _merge_top16(lane, vs[2], vi[2], vs[3], vi[3])
        tqs, tqi, bq = _merge_top16(lane, t01s, t01i, t23s, t23i)
        ts, ti, bt = _merge_top16(lane, tqs, tqi, vs[4], vi[4])
        thr_g = jnp.maximum(jnp.maximum(b01, b23), jnp.maximum(bq, bt))
        thrmax = jnp.maximum(thrmax, thr_g)

        li = ti - (base + goff)
        vno = li >> 4
        lno = li - (vno << 4)
        outs = []
        for c_v in (y1_v, x1_v, y2_v, x2_v):
            r = jnp.zeros((16,), jnp.float32)
            for k in range(5):
                ck = c_v[pl.ds(goff + k * 16, 16)]
                r = jnp.where(vno == k, ck[lno], r)
            outs.append(r)

        cs_v[pl.ds(g * 16, 16)] = ts
        ci_v[pl.ds(g * 16, 16)] = ti
        cy1_v[pl.ds(g * 16, 16)] = outs[0]
        cx1_v[pl.ds(g * 16, 16)] = outs[1]
        cy2_v[pl.ds(g * 16, 16)] = outs[2]
        cx2_v[pl.ds(g * 16, 16)] = outs[3]
        return thrmax

    thrmax = lax.fori_loop(0, _NG, _group, jnp.full((16,), neg))
    thr_v[...] = thrmax

    pltpu.sync_copy(cs_v, cs_out.at[pl.ds(wid * _M, _M)])
    pltpu.sync_copy(ci_v, ci_out.at[pl.ds(wid * _M, _M)])
    pltpu.sync_copy(cy1_v, cy1_out.at[pl.ds(wid * _M, _M)])
    pltpu.sync_copy(cx1_v, cx1_out.at[pl.ds(wid * _M, _M)])
    pltpu.sync_copy(cy2_v, cy2_out.at[pl.ds(wid * _M, _M)])
    pltpu.sync_copy(cx2_v, cx2_out.at[pl.ds(wid * _M, _M)])
    pltpu.sync_copy(thr_v, thr_out.at[pl.ds(wid * 16, 16)])

  return _sc_sel


_sc_select_cached = None


def _sc_select(*args):
    global _sc_select_cached
    if _sc_select_cached is None:
        _sc_select_cached = _build_sc_select()
    return _sc_select_cached(*args)


def _scal_iou_ge(by1a, bx1a, by2a, bx2a, bara, by1b, bx1b, by2b, bx2b, barb):
    yy1 = jnp.maximum(by1a, by1b)
    xx1 = jnp.maximum(bx1a, bx1b)
    yy2 = jnp.minimum(by2a, by2b)
    xx2 = jnp.minimum(bx2a, bx2b)
    inter = jnp.maximum(yy2 - yy1, 0.0) * jnp.maximum(xx2 - xx1, 0.0)
    union = bara + barb - inter
    iou = jnp.where(union > 0.0, inter / union, 0.0)
    return iou >= 0.7


def _nms_cand_body(mos_ref, b0_ref, thr_ref, s_ref, idxf_ref,
                   y1_ref, x1_ref, y2_ref, x2_ref,
                   out_ref, flag_ref, sw_ref, ar_ref, acc_ref):
    sw_ref[...] = s_ref[...]
    ar_ref[...] = (jnp.maximum(y2_ref[...] - y1_ref[...], 0.0)
                   * jnp.maximum(x2_ref[...] - x1_ref[...], 0.0))
    mos = mos_ref[0]
    flag_ref[0] = (mos != _MAXO).astype(jnp.int32)
    mthr = jnp.max(thr_ref[...])
    b0y1 = b0_ref[0]
    b0x1 = b0_ref[1]
    b0y2 = b0_ref[2]
    b0x2 = b0_ref[3]
    out_ref[:, pl.ds(0, 1)] = jnp.full((_MAXO + 8, 1), b0y1)
    out_ref[:, pl.ds(1, 1)] = jnp.full((_MAXO + 8, 1), b0x1)
    out_ref[:, pl.ds(2, 1)] = jnp.full((_MAXO + 8, 1), b0y2)
    out_ref[:, pl.ds(3, 1)] = jnp.full((_MAXO + 8, 1), b0x2)
    row8 = lax.broadcasted_iota(jnp.int32, (8, 4), 0)
    col8 = lax.broadcasted_iota(jnp.int32, (8, 4), 1)

    neg = jnp.float32(_NEG)
    big = jnp.float32(_NP)

    def body(i, rows):
        acc_ref[0] = 0

        @pl.when(rows < mos)
        def _():
            s = sw_ref[...]
            idxp = idxf_ref[...]
            y1 = y1_ref[...]
            x1 = x1_ref[...]
            y2 = y2_ref[...]
            x2 = x2_ref[...]
            ar = ar_ref[...]

            m1 = jnp.max(s)
            q1 = s == m1
            s2 = jnp.where(q1, neg, s)
            m2 = jnp.max(s2)
            q2 = s2 == m2
            s3 = jnp.where(q2, neg, s2)
            m3 = jnp.max(s3)
            q3 = s3 == m3
            s4 = jnp.where(q3, neg, s3)
            m4 = jnp.max(s4)
            q4 = s4 == m4
            s5 = jnp.where(q4, neg, s4)
            m5 = jnp.max(s5)
            q5 = s5 == m5
            s6 = jnp.where(q5, neg, s5)
            m6 = jnp.max(s6)
            q6 = s6 == m6

            idx1 = jnp.min(jnp.where(q1, idxp, big))
            sel1 = idxp == idx1
            ones = jnp.float32(1.0)
            cnt1 = jnp.sum(jnp.where(q1, ones, 0.0))
            cnt2 = jnp.sum(jnp.where(q2, ones, 0.0))
            cnt3 = jnp.sum(jnp.where(q3, ones, 0.0))
            cnt4 = jnp.sum(jnp.where(q4, ones, 0.0))
            cnt5 = jnp.sum(jnp.where(q5, ones, 0.0))
            cnt6 = jnp.sum(jnp.where(q6, ones, 0.0))

            def extract(sel):
                ey1 = jnp.max(jnp.where(sel, y1, neg))
                ex1 = jnp.max(jnp.where(sel, x1, neg))
                ey2 = jnp.max(jnp.where(sel, y2, neg))
                ex2 = jnp.max(jnp.where(sel, x2, neg))
                ear = (jnp.maximum(ey2 - ey1, 0.0)
                       * jnp.maximum(ex2 - ex1, 0.0))
                return (ey1, ex1, ey2, ex2, ear)

            c1 = extract(sel1)
            c2 = extract(q2)
            c3 = extract(q3)
            c4 = extract(q4)
            c5 = extract(q5)
            c6 = extract(q6)

            ok2 = (cnt1 <= 1.0) & (cnt2 <= 1.0) & (m2 > _NEG / 2.0)
            ok3 = ok2 & (cnt3 <= 1.0) & (m3 > _NEG / 2.0)
            ok4 = ok3 & (cnt4 <= 1.0) & (m4 > _NEG / 2.0)
            ok5 = ok4 & (cnt5 <= 1.0) & (m5 > _NEG / 2.0)
            ok6 = ok5 & (cnt6 <= 1.0) & (m6 > _NEG / 2.0)


            a2 = ok2 & ~_scal_iou_ge(*c1, *c2)
            a3 = ok3 & ~((a2 & _scal_iou_ge(*c2, *c3))
                         | _scal_iou_ge(*c1, *c3))
            a4 = ok4 & ~((a3 & _scal_iou_ge(*c3, *c4))
                         | (a2 & _scal_iou_ge(*c2, *c4))
                         | _scal_iou_ge(*c1, *c4))
            a5 = ok5 & ~((a4 & _scal_iou_ge(*c4, *c5))
                         | (a3 & _scal_iou_ge(*c3, *c5))
                         | (a2 & _scal_iou_ge(*c2, *c5))
                         | _scal_iou_ge(*c1, *c5))
            a6 = ok6 & ~((a5 & _scal_iou_ge(*c5, *c6))
                         | (a4 & _scal_iou_ge(*c4, *c6))
                         | (a3 & _scal_iou_ge(*c3, *c6))
                         | (a2 & _scal_iou_ge(*c2, *c6))
                         | _scal_iou_ge(*c1, *c6))

            r1 = rows
            r2 = r1 + 1
            r3 = r2 + jnp.where(a2, 1, 0)
            r4 = r3 + jnp.where(a3, 1, 0)
            r5 = r4 + jnp.where(a4, 1, 0)
            r6 = r5 + jnp.where(a5, 1, 0)
            w1 = r1 < mos
            w2 = a2 & (r2 < mos)
            w3 = a3 & (r3 < mos)
            w4 = a4 & (r4 < mos)
            w5 = a5 & (r5 < mos)
            w6 = a6 & (r6 < mos)

            mlast = jnp.where(w6, m6, jnp.where(w5, m5,
                    jnp.where(w4, m4, jnp.where(w3, m3,
                              jnp.where(w2, m2, m1)))))
            flag_ref[0] = flag_ref[0] | (mlast <= mthr).astype(jnp.int32) \
                | (m1 <= mthr).astype(jnp.int32)

            blk = jnp.where(col8 == 0, b0y1,
                  jnp.where(col8 == 1, b0x1,
                  jnp.where(col8 == 2, b0y2, b0x2)))
            for w, r, c in ((w1, r1, c1), (w2, r2, c2),
                            (w3, r3, c3), (w4, r4, c4),
                            (w5, r5, c5), (w6, r6, c6)):
                hit = w & (row8 == (r - rows))
                blk = jnp.where(hit & (col8 == 0), c[0],
                      jnp.where(hit & (col8 == 1), c[1],
                      jnp.where(hit & (col8 == 2), c[2],
                      jnp.where(hit & (col8 == 3), c[3], blk))))
            out_ref[pl.ds(rows, 8), :] = blk

            def supp_mask(c, selq):
                yy1 = jnp.maximum(c[0], y1)
                xx1 = jnp.maximum(c[1], x1)
                yy2 = jnp.minimum(c[2], y2)
                xx2 = jnp.minimum(c[3], x2)
                inter = (jnp.maximum(yy2 - yy1, 0.0)
                         * jnp.maximum(xx2 - xx1, 0.0))
                union = c[4] + ar - inter
                iou = jnp.where(union > 0.0, inter / union, 0.0)
                return (iou >= 0.7) | selq

            sm = supp_mask(c1, sel1)
            sm = sm | (w2 & supp_mask(c2, q2))
            sm = sm | (w3 & supp_mask(c3, q3))
            sm = sm | (w4 & supp_mask(c4, q4))
            sm = sm | (w5 & supp_mask(c5, q5))
            sm = sm | (w6 & supp_mask(c6, q6))
            sw_ref[...] = jnp.where(sm, neg, s)

            acc_ref[0] = (jnp.where(w1, 1, 0) + jnp.where(w2, 1, 0)
                          + jnp.where(w3, 1, 0) + jnp.where(w4, 1, 0)
                          + jnp.where(w5, 1, 0) + jnp.where(w6, 1, 0))

        return rows + acc_ref[0]

    rows = lax.fori_loop(0, 224, body, jnp.int32(0))
    flag_ref[0] = flag_ref[0] | (rows < mos).astype(jnp.int32)


def _nms_full_body(mos_ref, s_ref, y1_ref, x1_ref, y2_ref, x2_ref,
                   out_ref, sw_ref, ar_ref):
    sw_ref[...] = s_ref[...]
    ar_ref[...] = (jnp.maximum(y2_ref[...] - y1_ref[...], 0.0)
                   * jnp.maximum(x2_ref[...] - x1_ref[...], 0.0))
    mos = mos_ref[0]
    rows = lax.broadcasted_iota(jnp.int32, (_R, _C), 0)
    cols = lax.broadcasted_iota(jnp.int32, (_R, _C), 1)
    idxg = (rows * _C + cols).astype(jnp.float32)

    b0y1 = y1_ref[0, 0]
    b0x1 = x1_ref[0, 0]
    b0y2 = y2_ref[0, 0]
    b0x2 = x2_ref[0, 0]

    def body(i, _):
        s = sw_ref[...]
        m = jnp.max(s)
        idx = jnp.min(jnp.where(s == m, idxg, jnp.float32(_NP)))
        valid = (m > _NEG / 2.0) & (i < mos)
        sel = idxg == idx
        by1 = jnp.max(jnp.where(sel, y1_ref[...], _NEG))
        bx1 = jnp.max(jnp.where(sel, x1_ref[...], _NEG))
        by2 = jnp.max(jnp.where(sel, y2_ref[...], _NEG))
        bx2 = jnp.max(jnp.where(sel, x2_ref[...], _NEG))
        barea = jnp.max(jnp.where(sel, ar_ref[...], _NEG))

        yy1 = jnp.maximum(by1, y1_ref[...])
        xx1 = jnp.maximum(bx1, x1_ref[...])
        yy2 = jnp.minimum(by2, y2_ref[...])
        xx2 = jnp.minimum(bx2, x2_ref[...])
        inter = jnp.maximum(yy2 - yy1, 0.0) * jnp.maximum(xx2 - xx1, 0.0)
        union = barea + ar_ref[...] - inter
        iou = jnp.where(union > 0.0, inter / union, 0.0)
        supp = (iou >= 0.7) | sel
        sw_ref[...] = jnp.where(supp, _NEG, s)

        oy1 = jnp.where(valid, by1, b0y1)
        ox1 = jnp.where(valid, bx1, b0x1)
        oy2 = jnp.where(valid, by2, b0y2)
        ox2 = jnp.where(valid, bx2, b0x2)
        out_ref[pl.ds(i, 1), pl.ds(0, 1)] = jnp.full((1, 1), oy1)
        out_ref[pl.ds(i, 1), pl.ds(1, 1)] = jnp.full((1, 1), ox1)
        out_ref[pl.ds(i, 1), pl.ds(2, 1)] = jnp.full((1, 1), oy2)
        out_ref[pl.ds(i, 1), pl.ds(3, 1)] = jnp.full((1, 1), ox2)
        return 0

    lax.fori_loop(0, _MAXO, body, 0)


def kernel(rois, scores, max_output_size):
    s = jnp.squeeze(scores, axis=-1)
    s_p = jnp.concatenate([s, jnp.full((_NP - _N,), _NEG, jnp.float32)])
    zpad = jnp.zeros((_NP - _N,), jnp.float32)
    y1 = jnp.concatenate([rois[:, 0], zpad])
    x1 = jnp.concatenate([rois[:, 1], zpad])
    y2 = jnp.concatenate([rois[:, 2], zpad])
    x2 = jnp.concatenate([rois[:, 3], zpad])
    mos = jnp.asarray(max_output_size, jnp.int32).reshape(1)
    b0 = rois[0]

    cs, ci, cy1, cx1, cy2, cx2, thr = _sc_select(s_p, y1, x1, y2, x2)

    vspec = pl.BlockSpec(memory_space=pltpu.VMEM)
    sspec = pl.BlockSpec(memory_space=pltpu.SMEM)
    fast_out, flag = pl.pallas_call(
        _nms_cand_body,
        out_shape=[jax.ShapeDtypeStruct((_MAXO + 8, 4), jnp.float32),
                   jax.ShapeDtypeStruct((1,), jnp.int32)],
        in_specs=[sspec, sspec] + [vspec] * 7,
        out_specs=[vspec, sspec],
        scratch_shapes=[pltpu.VMEM((_CR, _CC), jnp.float32),
                        pltpu.VMEM((_CR, _CC), jnp.float32),
                        pltpu.SMEM((1,), jnp.int32)],
    )(mos, b0, thr.reshape(8, -1), cs.reshape(_CR, _CC),
      ci.astype(jnp.float32).reshape(_CR, _CC), cy1.reshape(_CR, _CC),
      cx1.reshape(_CR, _CC), cy2.reshape(_CR, _CC), cx2.reshape(_CR, _CC))

    def _full(_):
        return pl.pallas_call(
            _nms_full_body,
            out_shape=jax.ShapeDtypeStruct((_MAXO, 4), jnp.float32),
            in_specs=[sspec] + [vspec] * 5,
            scratch_shapes=[pltpu.VMEM((_R, _C), jnp.float32),
                            pltpu.VMEM((_R, _C), jnp.float32)],
        )(mos, s_p.reshape(_R, _C), y1.reshape(_R, _C), x1.reshape(_R, _C),
          y2.reshape(_R, _C), x2.reshape(_R, _C))

    def _fast(_):
        return fast_out[:_MAXO]

    return lax.cond(flag[0] > 0, _full, _fast, None)
